# Initial kernel scaffold; baseline (speedup 1.0000x reference)
import jax
import jax.numpy as jnp
from jax.experimental import pallas as pl


def kernel(x, edge_index, subg_nodeidx, subg_nodelabel, subg_edge_index, batch, W_enc, b_enc, label_emb, W_l, b_l, W_g, b_g, W_out, b_out):
    raise NotImplementedError("write your pallas kernel here")


# speedup vs baseline: 1.0408x; 1.0408x over previous
import functools

import jax
import jax.numpy as jnp
from jax import lax
from jax.experimental import pallas as pl
from jax.experimental.pallas import tpu as pltpu

_N = 10000
_E = 320000
_NSUB = 40000
_ESUB = 320000
_NGRAPH = 64
_EMB = 128

_BM = 2000


def _mm_relu_body(n_parts, residual, *refs):
    *x_refs, w_ref, b_ref, o_ref = refs
    acc = x_refs[0][...]
    for r in x_refs[1:n_parts]:
        acc = acc + r[...]
    y = jnp.dot(acc, w_ref[...], preferred_element_type=jnp.float32)
    y = jnp.maximum(y + b_ref[...], 0.0)
    if residual:
        y = x_refs[n_parts][...] + y
    o_ref[...] = y


def _mm_relu(parts, w, b, residual=None):
    m = parts[0].shape[0]
    k = parts[0].shape[1]
    n = w.shape[1]
    inputs = list(parts) + ([residual] if residual is not None else [])
    grid = m // _BM
    body = functools.partial(_mm_relu_body, len(parts), residual is not None)
    return pl.pallas_call(
        body,
        grid=(grid,),
        in_specs=[pl.BlockSpec((_BM, k), lambda i: (i, 0)) for _ in inputs]
        + [
            pl.BlockSpec((k, n), lambda i: (0, 0)),
            pl.BlockSpec((n,), lambda i: (0,)),
        ],
        out_specs=pl.BlockSpec((_BM, n), lambda i: (i, 0)),
        out_shape=jax.ShapeDtypeStruct((m, n), jnp.float32),
    )(*inputs, w, b)


def _mean_residual_body(h_ref, s0_ref, s1_ref, c0_ref, c1_ref, o_ref):
    cnt = c0_ref[...] + c1_ref[...]
    cnt = jnp.maximum(cnt[:, 0:1], 1.0)
    o_ref[...] = h_ref[...] + (s0_ref[...] + s1_ref[...]) / cnt


def _mean_residual(h, s0, s1, c0, c1):
    m, n = h.shape
    grid = m // _BM
    return pl.pallas_call(
        _mean_residual_body,
        grid=(grid,),
        in_specs=[
            pl.BlockSpec((_BM, n), lambda i: (i, 0)),
            pl.BlockSpec((_BM, n), lambda i: (i, 0)),
            pl.BlockSpec((_BM, n), lambda i: (i, 0)),
            pl.BlockSpec((_BM, 16), lambda i: (i, 0)),
            pl.BlockSpec((_BM, 16), lambda i: (i, 0)),
        ],
        out_specs=pl.BlockSpec((_BM, n), lambda i: (i, 0)),
        out_shape=jax.ShapeDtypeStruct((m, n), jnp.float32),
    )(h, s0, s1, c0, c1)


def _pool_body(h_ref, batch_ref, w_ref, b_ref, o_ref, acc_ref, cnt_ref):
    i = pl.program_id(0)

    @pl.when(i == 0)
    def _():
        acc_ref[...] = jnp.zeros_like(acc_ref)
        cnt_ref[...] = jnp.zeros_like(cnt_ref)

    seg = batch_ref[...].reshape(1, _BM)
    gids = lax.broadcasted_iota(jnp.int32, (_NGRAPH, _BM), 0)
    mask = (gids == seg).astype(jnp.float32)
    acc_ref[...] += jnp.dot(mask, h_ref[...], preferred_element_type=jnp.float32)
    cnt_ref[...] += jnp.sum(mask, axis=1, keepdims=True)

    @pl.when(i == pl.num_programs(0) - 1)
    def _():
        hg = acc_ref[...] / jnp.maximum(cnt_ref[...], 1.0)
        o_ref[...] = jnp.dot(hg, w_ref[...], preferred_element_type=jnp.float32) + b_ref[...]


def _pool_predict(h, batch, w_out, b_out):
    m, n = h.shape
    ntask = w_out.shape[1]
    w_pad = jnp.zeros((n, 128), jnp.float32).at[:, :ntask].set(w_out)
    b_pad = jnp.zeros((128,), jnp.float32).at[:ntask].set(b_out)
    batch3 = batch.reshape(m // _BM, 1, _BM)
    grid = m // _BM
    out = pl.pallas_call(
        _pool_body,
        grid=(grid,),
        in_specs=[
            pl.BlockSpec((_BM, n), lambda i: (i, 0)),
            pl.BlockSpec((1, 1, _BM), lambda i: (i, 0, 0)),
            pl.BlockSpec((n, 128), lambda i: (0, 0)),
            pl.BlockSpec((128,), lambda i: (0,)),
        ],
        out_specs=pl.BlockSpec((_NGRAPH, 128), lambda i: (0, 0)),
        out_shape=jax.ShapeDtypeStruct((_NGRAPH, 128), jnp.float32),
        scratch_shapes=[
            pltpu.VMEM((_NGRAPH, n), jnp.float32),
            pltpu.VMEM((_NGRAPH, 1), jnp.float32),
        ],
    )(h, batch3, w_pad, b_pad)
    return out[:, :ntask]


def kernel(x, edge_index, subg_nodeidx, subg_nodelabel, subg_edge_index, batch,
           W_enc, b_enc, label_emb, W_l, b_l, W_g, b_g, W_out, b_out):
    h = _mm_relu([x], W_enc, b_enc)

    xs = jnp.take(h, subg_nodeidx, axis=0) + jnp.take(label_emb, subg_nodelabel, axis=0)

    src, dst = subg_edge_index[0], subg_edge_index[1]
    agg = jax.ops.segment_sum(jnp.take(xs, src, axis=0), dst, num_segments=_NSUB)

    h_sub = _mm_relu([agg], W_l, b_l)

    sums = jax.ops.segment_sum(h_sub, subg_nodeidx, num_segments=_N)
    cnts = jax.ops.segment_sum(jnp.ones((_NSUB,), jnp.float32), subg_nodeidx,
                               num_segments=_N)
    c16 = jnp.broadcast_to(cnts[:, None], (_N, 16))
    z = jnp.zeros_like(sums)
    z16 = jnp.zeros_like(c16)
    h1 = _mean_residual(h, sums, z, c16, z16)

    gsrc, gdst = edge_index[0], edge_index[1]
    gagg = jax.ops.segment_sum(jnp.take(h1, gsrc, axis=0), gdst, num_segments=_N)

    h2 = _mm_relu([gagg], W_g, b_g, residual=h1)

    return _pool_predict(h2, batch, W_out, b_out)


# speedup vs baseline: 1.5357x; 1.4755x over previous
import functools

import jax
import jax.numpy as jnp
from jax import lax
from jax.experimental import pallas as pl
from jax.experimental.pallas import tpu as pltpu
from jax.experimental.pallas import tpu_sc as plsc

_N = 10000
_E = 320000
_NSUB = 40000
_ESUB = 320000
_NGRAPH = 64
_EMB = 128

_BM = 2000


def _mm_relu_body(n_parts, residual, *refs):
    *x_refs, w_ref, b_ref, o_ref = refs
    acc = x_refs[0][...]
    for r in x_refs[1:n_parts]:
        acc = acc + r[...]
    y = jnp.dot(acc, w_ref[...], preferred_element_type=jnp.float32)
    y = jnp.maximum(y + b_ref[...], 0.0)
    if residual:
        y = x_refs[n_parts][...] + y
    o_ref[...] = y


def _mm_relu(parts, w, b, residual=None):
    m = parts[0].shape[0]
    k = parts[0].shape[1]
    n = w.shape[1]
    inputs = list(parts) + ([residual] if residual is not None else [])
    grid = m // _BM
    body = functools.partial(_mm_relu_body, len(parts), residual is not None)
    return pl.pallas_call(
        body,
        grid=(grid,),
        in_specs=[pl.BlockSpec((_BM, k), lambda i: (i, 0)) for _ in inputs]
        + [
            pl.BlockSpec((k, n), lambda i: (0, 0)),
            pl.BlockSpec((n,), lambda i: (0,)),
        ],
        out_specs=pl.BlockSpec((_BM, n), lambda i: (i, 0)),
        out_shape=jax.ShapeDtypeStruct((m, n), jnp.float32),
    )(*inputs, w, b)


def _mean_residual_body(h_ref, s0_ref, s1_ref, c0_ref, c1_ref, o_ref):
    cnt = c0_ref[...] + c1_ref[...]
    cnt = jnp.maximum(cnt[:, 0:1], 1.0)
    o_ref[...] = h_ref[...] + (s0_ref[...] + s1_ref[...]) / cnt


def _mean_residual(h, s0, s1, c0, c1):
    m, n = h.shape
    grid = m // _BM
    return pl.pallas_call(
        _mean_residual_body,
        grid=(grid,),
        in_specs=[
            pl.BlockSpec((_BM, n), lambda i: (i, 0)),
            pl.BlockSpec((_BM, n), lambda i: (i, 0)),
            pl.BlockSpec((_BM, n), lambda i: (i, 0)),
            pl.BlockSpec((_BM, 16), lambda i: (i, 0)),
            pl.BlockSpec((_BM, 16), lambda i: (i, 0)),
        ],
        out_specs=pl.BlockSpec((_BM, n), lambda i: (i, 0)),
        out_shape=jax.ShapeDtypeStruct((m, n), jnp.float32),
    )(h, s0, s1, c0, c1)


def _pool_body(h_ref, batch_ref, w_ref, b_ref, o_ref, acc_ref, cnt_ref):
    i = pl.program_id(0)

    @pl.when(i == 0)
    def _():
        acc_ref[...] = jnp.zeros_like(acc_ref)
        cnt_ref[...] = jnp.zeros_like(cnt_ref)

    seg = batch_ref[...].reshape(1, _BM)
    gids = lax.broadcasted_iota(jnp.int32, (_NGRAPH, _BM), 0)
    mask = (gids == seg).astype(jnp.float32)
    acc_ref[...] += jnp.dot(mask, h_ref[...], preferred_element_type=jnp.float32)
    cnt_ref[...] += jnp.sum(mask, axis=1, keepdims=True)

    @pl.when(i == pl.num_programs(0) - 1)
    def _():
        hg = acc_ref[...] / jnp.maximum(cnt_ref[...], 1.0)
        o_ref[...] = jnp.dot(hg, w_ref[...], preferred_element_type=jnp.float32) + b_ref[...]


def _pool_predict(h, batch, w_out, b_out):
    m, n = h.shape
    ntask = w_out.shape[1]
    w_pad = jnp.zeros((n, 128), jnp.float32).at[:, :ntask].set(w_out)
    b_pad = jnp.zeros((128,), jnp.float32).at[:ntask].set(b_out)
    batch3 = batch.reshape(m // _BM, 1, _BM)
    grid = m // _BM
    out = pl.pallas_call(
        _pool_body,
        grid=(grid,),
        in_specs=[
            pl.BlockSpec((_BM, n), lambda i: (i, 0)),
            pl.BlockSpec((1, 1, _BM), lambda i: (i, 0, 0)),
            pl.BlockSpec((n, 128), lambda i: (0, 0)),
            pl.BlockSpec((128,), lambda i: (0,)),
        ],
        out_specs=pl.BlockSpec((_NGRAPH, 128), lambda i: (0, 0)),
        out_shape=jax.ShapeDtypeStruct((_NGRAPH, 128), jnp.float32),
        scratch_shapes=[
            pltpu.VMEM((_NGRAPH, n), jnp.float32),
            pltpu.VMEM((_NGRAPH, 1), jnp.float32),
        ],
    )(h, batch3, w_pad, b_pad)
    return out[:, :ntask]


_SC_NC = 2
_SC_NS = 16
_SC_G = 80


def _sc_mesh():
    return plsc.VectorSubcoreMesh(core_axis_name="c", subcore_axis_name="s")


def _sc_segsum_edges(h, src, dst, zeros):
    n, emb = h.shape
    e = src.shape[0]
    e_per_w = e // (_SC_NC * _SC_NS)
    n_groups = e_per_w // _SC_G
    stripe = ((n // _SC_NS + 7) // 8) * 8
    last = n - stripe * (_SC_NS - 1)
    z2 = zeros[: stripe * emb].reshape(stripe, emb)

    def body(h_hbm, src_hbm, dst_hbm, z_hbm, out_hbm, acc, idxs, idxd, vals):
        c = lax.axis_index("c")
        s = lax.axis_index("s")
        wid = c * _SC_NS + s

        @pl.when(s < _SC_NS - 1)
        def _():
            pltpu.sync_copy(z_hbm, acc.at[pl.ds(s * stripe, stripe)])

        @pl.when(s == _SC_NS - 1)
        def _():
            pltpu.sync_copy(z_hbm.at[pl.ds(0, last)],
                            acc.at[pl.ds(s * stripe, last)])

        plsc.subcore_barrier()
        base0 = wid * e_per_w

        def step(g, carry):
            base = base0 + g * _SC_G
            pltpu.sync_copy(src_hbm.at[pl.ds(base, _SC_G)], idxs)
            pltpu.sync_copy(dst_hbm.at[pl.ds(base, _SC_G)], idxd)
            pltpu.sync_copy(h_hbm.at[idxs], vals)
            pltpu.sync_copy(vals, acc.at[idxd], add=True)
            return carry

        lax.fori_loop(0, n_groups, step, 0)
        plsc.subcore_barrier()

        @pl.when(s < _SC_NS - 1)
        def _():
            pltpu.sync_copy(acc.at[pl.ds(s * stripe, stripe)],
                            out_hbm.at[c, pl.ds(s * stripe, stripe)])

        @pl.when(s == _SC_NS - 1)
        def _():
            pltpu.sync_copy(acc.at[pl.ds(s * stripe, last)],
                            out_hbm.at[c, pl.ds(s * stripe, last)])

    f = pl.kernel(
        body,
        out_type=jax.ShapeDtypeStruct((_SC_NC, n, emb), jnp.float32),
        mesh=_sc_mesh(),
        scratch_types=[
            pltpu.VMEM_SHARED((n, emb), jnp.float32),
            pltpu.VMEM((_SC_G,), jnp.int32),
            pltpu.VMEM((_SC_G,), jnp.int32),
            pltpu.VMEM((_SC_G, emb), jnp.float32),
        ],
    )
    return f(h, src, dst, z2)


def kernel(x, edge_index, subg_nodeidx, subg_nodelabel, subg_edge_index, batch,
           W_enc, b_enc, label_emb, W_l, b_l, W_g, b_g, W_out, b_out):
    h = _mm_relu([x], W_enc, b_enc)

    xs = jnp.take(h, subg_nodeidx, axis=0) + jnp.take(label_emb, subg_nodelabel, axis=0)

    src, dst = subg_edge_index[0], subg_edge_index[1]
    agg = jax.ops.segment_sum(jnp.take(xs, src, axis=0), dst, num_segments=_NSUB)

    h_sub = _mm_relu([agg], W_l, b_l)

    sums = jax.ops.segment_sum(h_sub, subg_nodeidx, num_segments=_N)
    cnts = jax.ops.segment_sum(jnp.ones((_NSUB,), jnp.float32), subg_nodeidx,
                               num_segments=_N)
    c16 = jnp.broadcast_to(cnts[:, None], (_N, 16))
    z = jnp.zeros_like(sums)
    z16 = jnp.zeros_like(c16)
    h1 = _mean_residual(h, sums, z, c16, z16)

    zeros = jnp.zeros((81920,), jnp.float32)
    gp = _sc_segsum_edges(h1, edge_index[0], edge_index[1], zeros)

    h2 = _mm_relu([gp[0], gp[1]], W_g, b_g, residual=h1)

    return _pool_predict(h2, batch, W_out, b_out)


# speedup vs baseline: 1.6230x; 1.0569x over previous
import functools

import jax
import jax.numpy as jnp
from jax import lax
from jax.experimental import pallas as pl
from jax.experimental.pallas import tpu as pltpu
from jax.experimental.pallas import tpu_sc as plsc

_N = 10000
_E = 320000
_NSUB = 40000
_ESUB = 320000
_NGRAPH = 64
_EMB = 128

_BM = 2000


def _mm_relu_body(n_parts, residual, *refs):
    *x_refs, w_ref, b_ref, o_ref = refs
    acc = x_refs[0][...]
    for r in x_refs[1:n_parts]:
        acc = acc + r[...]
    y = jnp.dot(acc, w_ref[...], preferred_element_type=jnp.float32)
    y = jnp.maximum(y + b_ref[...], 0.0)
    if residual:
        y = x_refs[n_parts][...] + y
    o_ref[...] = y


def _mm_relu(parts, w, b, residual=None):
    m = parts[0].shape[0]
    k = parts[0].shape[1]
    n = w.shape[1]
    inputs = list(parts) + ([residual] if residual is not None else [])
    grid = m // _BM
    body = functools.partial(_mm_relu_body, len(parts), residual is not None)
    return pl.pallas_call(
        body,
        grid=(grid,),
        in_specs=[pl.BlockSpec((_BM, k), lambda i: (i, 0)) for _ in inputs]
        + [
            pl.BlockSpec((k, n), lambda i: (0, 0)),
            pl.BlockSpec((n,), lambda i: (0,)),
        ],
        out_specs=pl.BlockSpec((_BM, n), lambda i: (i, 0)),
        out_shape=jax.ShapeDtypeStruct((m, n), jnp.float32),
    )(*inputs, w, b)


def _mean_residual_body(h_ref, s0_ref, s1_ref, c0_ref, c1_ref, o_ref):
    cnt = jnp.maximum((c0_ref[...] + c1_ref[...])[:, 0:1], 1.0)
    o_ref[...] = h_ref[...] + (s0_ref[...] + s1_ref[...]) / cnt


def _mean_residual(h, s0, s1, c0, c1):
    m, n = h.shape
    grid = m // _BM
    return pl.pallas_call(
        _mean_residual_body,
        grid=(grid,),
        in_specs=[pl.BlockSpec((_BM, n), lambda i: (i, 0)) for _ in range(5)],
        out_specs=pl.BlockSpec((_BM, n), lambda i: (i, 0)),
        out_shape=jax.ShapeDtypeStruct((m, n), jnp.float32),
    )(h, s0, s1, c0, c1)


def _pool_body(h_ref, batch_ref, w_ref, b_ref, o_ref, acc_ref, cnt_ref):
    i = pl.program_id(0)

    @pl.when(i == 0)
    def _():
        acc_ref[...] = jnp.zeros_like(acc_ref)
        cnt_ref[...] = jnp.zeros_like(cnt_ref)

    seg = batch_ref[...].reshape(1, _BM)
    gids = lax.broadcasted_iota(jnp.int32, (_NGRAPH, _BM), 0)
    mask = (gids == seg).astype(jnp.float32)
    acc_ref[...] += jnp.dot(mask, h_ref[...], preferred_element_type=jnp.float32)
    cnt_ref[...] += jnp.sum(mask, axis=1, keepdims=True)

    @pl.when(i == pl.num_programs(0) - 1)
    def _():
        hg = acc_ref[...] / jnp.maximum(cnt_ref[...], 1.0)
        o_ref[...] = jnp.dot(hg, w_ref[...], preferred_element_type=jnp.float32) + b_ref[...]


def _pool_predict(h, batch, w_out, b_out):
    m, n = h.shape
    ntask = w_out.shape[1]
    w_pad = jnp.zeros((n, 128), jnp.float32).at[:, :ntask].set(w_out)
    b_pad = jnp.zeros((128,), jnp.float32).at[:ntask].set(b_out)
    batch3 = batch.reshape(m // _BM, 1, _BM)
    grid = m // _BM
    out = pl.pallas_call(
        _pool_body,
        grid=(grid,),
        in_specs=[
            pl.BlockSpec((_BM, n), lambda i: (i, 0)),
            pl.BlockSpec((1, 1, _BM), lambda i: (i, 0, 0)),
            pl.BlockSpec((n, 128), lambda i: (0, 0)),
            pl.BlockSpec((128,), lambda i: (0,)),
        ],
        out_specs=pl.BlockSpec((_NGRAPH, 128), lambda i: (0, 0)),
        out_shape=jax.ShapeDtypeStruct((_NGRAPH, 128), jnp.float32),
        scratch_shapes=[
            pltpu.VMEM((_NGRAPH, n), jnp.float32),
            pltpu.VMEM((_NGRAPH, 1), jnp.float32),
        ],
    )(h, batch3, w_pad, b_pad)
    return out[:, :ntask]


_SC_NC = 2
_SC_NS = 16
_SC_G = 80


def _sc_mesh():
    return plsc.VectorSubcoreMesh(core_axis_name="c", subcore_axis_name="s")


def _sc_segsum_edges(h, src, dst, zeros):
    n, emb = h.shape
    e = src.shape[0]
    e_per_w = e // (_SC_NC * _SC_NS)
    n_groups = e_per_w // _SC_G
    stripe = ((n // _SC_NS + 7) // 8) * 8
    last = n - stripe * (_SC_NS - 1)
    z2 = zeros[: stripe * emb].reshape(stripe, emb)

    def body(h_hbm, src_hbm, dst_hbm, z_hbm, out_hbm, acc, idxs, idxd, vals):
        c = lax.axis_index("c")
        s = lax.axis_index("s")
        wid = c * _SC_NS + s

        @pl.when(s < _SC_NS - 1)
        def _():
            pltpu.sync_copy(z_hbm, acc.at[pl.ds(s * stripe, stripe)])

        @pl.when(s == _SC_NS - 1)
        def _():
            pltpu.sync_copy(z_hbm.at[pl.ds(0, last)],
                            acc.at[pl.ds(s * stripe, last)])

        plsc.subcore_barrier()
        base0 = wid * e_per_w

        def step(g, carry):
            base = base0 + g * _SC_G
            pltpu.sync_copy(src_hbm.at[pl.ds(base, _SC_G)], idxs)
            pltpu.sync_copy(dst_hbm.at[pl.ds(base, _SC_G)], idxd)
            pltpu.sync_copy(h_hbm.at[idxs], vals)
            pltpu.sync_copy(vals, acc.at[idxd], add=True)
            return carry

        lax.fori_loop(0, n_groups, step, 0)
        plsc.subcore_barrier()

        @pl.when(s < _SC_NS - 1)
        def _():
            pltpu.sync_copy(acc.at[pl.ds(s * stripe, stripe)],
                            out_hbm.at[c, pl.ds(s * stripe, stripe)])

        @pl.when(s == _SC_NS - 1)
        def _():
            pltpu.sync_copy(acc.at[pl.ds(s * stripe, last)],
                            out_hbm.at[c, pl.ds(s * stripe, last)])

    f = pl.kernel(
        body,
        out_type=jax.ShapeDtypeStruct((_SC_NC, n, emb), jnp.float32),
        mesh=_sc_mesh(),
        scratch_types=[
            pltpu.VMEM_SHARED((n, emb), jnp.float32),
            pltpu.VMEM((_SC_G,), jnp.int32),
            pltpu.VMEM((_SC_G,), jnp.int32),
            pltpu.VMEM((_SC_G, emb), jnp.float32),
        ],
    )
    return f(h, src, dst, z2)


def _sc_scatter_mean(h_sub, nodeidx, zeros, n_out):
    nsub, emb = h_sub.shape
    n_groups = nsub // _SC_G
    k_max = (n_groups + _SC_NC * _SC_NS - 1) // (_SC_NC * _SC_NS)
    stripe = ((n_out // _SC_NS + 7) // 8) * 8
    last = n_out - stripe * (_SC_NS - 1)
    z_s = zeros[: stripe * emb].reshape(stripe, emb)
    ones = jnp.ones((_SC_G, emb), jnp.float32)

    def body(hs_hbm, ni_hbm, z_hbm, ones_hbm, sum_hbm, cnt_hbm,
             acc, idxd, vals, ones_v):
        c = lax.axis_index("c")
        s = lax.axis_index("s")
        wid = c * _SC_NS + s
        pltpu.sync_copy(ones_hbm, ones_v)

        def zero_acc():
            @pl.when(s < _SC_NS - 1)
            def _():
                pltpu.sync_copy(z_hbm, acc.at[pl.ds(s * stripe, stripe)])

            @pl.when(s == _SC_NS - 1)
            def _():
                pltpu.sync_copy(z_hbm.at[pl.ds(0, last)],
                                acc.at[pl.ds(s * stripe, last)])

        def flush_acc(dst_hbm):
            @pl.when(s < _SC_NS - 1)
            def _():
                pltpu.sync_copy(acc.at[pl.ds(s * stripe, stripe)],
                                dst_hbm.at[c, pl.ds(s * stripe, stripe)])

            @pl.when(s == _SC_NS - 1)
            def _():
                pltpu.sync_copy(acc.at[pl.ds(s * stripe, last)],
                                dst_hbm.at[c, pl.ds(s * stripe, last)])

        def scan_groups(do_group):
            def step(k, carry):
                g = wid * k_max + k

                @pl.when(g < n_groups)
                def _():
                    do_group(g * _SC_G)

                return carry

            lax.fori_loop(0, k_max, step, 0)

        zero_acc()
        plsc.subcore_barrier()

        def sum_group(base):
            pltpu.sync_copy(ni_hbm.at[pl.ds(base, _SC_G)], idxd)
            pltpu.sync_copy(hs_hbm.at[pl.ds(base, _SC_G)], vals)
            pltpu.sync_copy(vals, acc.at[idxd], add=True)

        scan_groups(sum_group)
        plsc.subcore_barrier()
        flush_acc(sum_hbm)
        plsc.subcore_barrier()

        zero_acc()
        plsc.subcore_barrier()

        def cnt_group(base):
            pltpu.sync_copy(ni_hbm.at[pl.ds(base, _SC_G)], idxd)
            pltpu.sync_copy(ones_v, acc.at[idxd], add=True)

        scan_groups(cnt_group)
        plsc.subcore_barrier()
        flush_acc(cnt_hbm)

    f = pl.kernel(
        body,
        out_type=(jax.ShapeDtypeStruct((_SC_NC, n_out, emb), jnp.float32),
                  jax.ShapeDtypeStruct((_SC_NC, n_out, emb), jnp.float32)),
        mesh=_sc_mesh(),
        scratch_types=[
            pltpu.VMEM_SHARED((n_out, emb), jnp.float32),
            pltpu.VMEM((_SC_G,), jnp.int32),
            pltpu.VMEM((_SC_G, emb), jnp.float32),
            pltpu.VMEM((_SC_G, emb), jnp.float32),
        ],
    )
    return f(h_sub, nodeidx, z_s, ones)


def kernel(x, edge_index, subg_nodeidx, subg_nodelabel, subg_edge_index, batch,
           W_enc, b_enc, label_emb, W_l, b_l, W_g, b_g, W_out, b_out):
    h = _mm_relu([x], W_enc, b_enc)

    xs = jnp.take(h, subg_nodeidx, axis=0) + jnp.take(label_emb, subg_nodelabel, axis=0)

    src, dst = subg_edge_index[0], subg_edge_index[1]
    agg = jax.ops.segment_sum(jnp.take(xs, src, axis=0), dst, num_segments=_NSUB)

    h_sub = _mm_relu([agg], W_l, b_l)

    zeros = jnp.zeros((81920,), jnp.float32)

    sums, cnts = _sc_scatter_mean(h_sub, subg_nodeidx, zeros, _N)
    h1 = _mean_residual(h, sums[0], sums[1], cnts[0], cnts[1])

    gp = _sc_segsum_edges(h1, edge_index[0], edge_index[1], zeros)

    h2 = _mm_relu([gp[0], gp[1]], W_g, b_g, residual=h1)

    return _pool_predict(h2, batch, W_out, b_out)


# speedup vs baseline: 2.5425x; 1.5665x over previous
import functools

import jax
import jax.numpy as jnp
from jax import lax
from jax.experimental import pallas as pl
from jax.experimental.pallas import tpu as pltpu
from jax.experimental.pallas import tpu_sc as plsc

_N = 10000
_E = 320000
_NSUB = 40000
_ESUB = 320000
_NGRAPH = 64
_EMB = 128

_BM = 2000


def _mm_relu_body(n_parts, residual, *refs):
    *x_refs, w_ref, b_ref, o_ref = refs
    acc = x_refs[0][...]
    for r in x_refs[1:n_parts]:
        acc = acc + r[...]
    y = jnp.dot(acc, w_ref[...], preferred_element_type=jnp.float32)
    y = jnp.maximum(y + b_ref[...], 0.0)
    if residual:
        y = x_refs[n_parts][...] + y
    o_ref[...] = y


def _mm_relu(parts, w, b, residual=None):
    m = parts[0].shape[0]
    k = parts[0].shape[1]
    n = w.shape[1]
    inputs = list(parts) + ([residual] if residual is not None else [])
    grid = m // _BM
    body = functools.partial(_mm_relu_body, len(parts), residual is not None)
    return pl.pallas_call(
        body,
        grid=(grid,),
        in_specs=[pl.BlockSpec((_BM, k), lambda i: (i, 0)) for _ in inputs]
        + [
            pl.BlockSpec((k, n), lambda i: (0, 0)),
            pl.BlockSpec((n,), lambda i: (0,)),
        ],
        out_specs=pl.BlockSpec((_BM, n), lambda i: (i, 0)),
        out_shape=jax.ShapeDtypeStruct((m, n), jnp.float32),
    )(*inputs, w, b)


def _mean_residual_body(h_ref, s0_ref, s1_ref, c0_ref, c1_ref, o_ref):
    cnt = jnp.maximum((c0_ref[...] + c1_ref[...])[:, 0:1], 1.0)
    o_ref[...] = h_ref[...] + (s0_ref[...] + s1_ref[...]) / cnt


def _mean_residual(h, s0, s1, c0, c1):
    m, n = h.shape
    grid = m // _BM
    return pl.pallas_call(
        _mean_residual_body,
        grid=(grid,),
        in_specs=[pl.BlockSpec((_BM, n), lambda i: (i, 0)) for _ in range(5)],
        out_specs=pl.BlockSpec((_BM, n), lambda i: (i, 0)),
        out_shape=jax.ShapeDtypeStruct((m, n), jnp.float32),
    )(h, s0, s1, c0, c1)


def _pool_body(h_ref, batch_ref, w_ref, b_ref, o_ref, acc_ref, cnt_ref):
    i = pl.program_id(0)

    @pl.when(i == 0)
    def _():
        acc_ref[...] = jnp.zeros_like(acc_ref)
        cnt_ref[...] = jnp.zeros_like(cnt_ref)

    seg = batch_ref[...].reshape(1, _BM)
    gids = lax.broadcasted_iota(jnp.int32, (_NGRAPH, _BM), 0)
    mask = (gids == seg).astype(jnp.float32)
    acc_ref[...] += jnp.dot(mask, h_ref[...], preferred_element_type=jnp.float32)
    cnt_ref[...] += jnp.sum(mask, axis=1, keepdims=True)

    @pl.when(i == pl.num_programs(0) - 1)
    def _():
        hg = acc_ref[...] / jnp.maximum(cnt_ref[...], 1.0)
        o_ref[...] = jnp.dot(hg, w_ref[...], preferred_element_type=jnp.float32) + b_ref[...]


def _pool_predict(h, batch, w_out, b_out):
    m, n = h.shape
    ntask = w_out.shape[1]
    w_pad = jnp.zeros((n, 128), jnp.float32).at[:, :ntask].set(w_out)
    b_pad = jnp.zeros((128,), jnp.float32).at[:ntask].set(b_out)
    batch3 = batch.reshape(m // _BM, 1, _BM)
    grid = m // _BM
    out = pl.pallas_call(
        _pool_body,
        grid=(grid,),
        in_specs=[
            pl.BlockSpec((_BM, n), lambda i: (i, 0)),
            pl.BlockSpec((1, 1, _BM), lambda i: (i, 0, 0)),
            pl.BlockSpec((n, 128), lambda i: (0, 0)),
            pl.BlockSpec((128,), lambda i: (0,)),
        ],
        out_specs=pl.BlockSpec((_NGRAPH, 128), lambda i: (0, 0)),
        out_shape=jax.ShapeDtypeStruct((_NGRAPH, 128), jnp.float32),
        scratch_shapes=[
            pltpu.VMEM((_NGRAPH, n), jnp.float32),
            pltpu.VMEM((_NGRAPH, 1), jnp.float32),
        ],
    )(h, batch3, w_pad, b_pad)
    return out[:, :ntask]


_SC_NC = 2
_SC_NS = 16
_SC_G = 80


def _sc_mesh():
    return plsc.VectorSubcoreMesh(core_axis_name="c", subcore_axis_name="s")


def _sc_segsum_edges(h, src, dst, zeros):
    n, emb = h.shape
    e = src.shape[0]
    e_per_w = e // (_SC_NC * _SC_NS)
    n_groups = e_per_w // _SC_G
    stripe = ((n // _SC_NS + 7) // 8) * 8
    last = n - stripe * (_SC_NS - 1)
    z2 = zeros[: stripe * emb].reshape(stripe, emb)

    def body(h_hbm, src_hbm, dst_hbm, z_hbm, out_hbm, acc, idxs, idxd, vals):
        c = lax.axis_index("c")
        s = lax.axis_index("s")
        wid = c * _SC_NS + s

        @pl.when(s < _SC_NS - 1)
        def _():
            pltpu.sync_copy(z_hbm, acc.at[pl.ds(s * stripe, stripe)])

        @pl.when(s == _SC_NS - 1)
        def _():
            pltpu.sync_copy(z_hbm.at[pl.ds(0, last)],
                            acc.at[pl.ds(s * stripe, last)])

        plsc.subcore_barrier()
        base0 = wid * e_per_w

        def step(g, carry):
            base = base0 + g * _SC_G
            pltpu.sync_copy(src_hbm.at[pl.ds(base, _SC_G)], idxs)
            pltpu.sync_copy(dst_hbm.at[pl.ds(base, _SC_G)], idxd)
            pltpu.sync_copy(h_hbm.at[idxs], vals)
            pltpu.sync_copy(vals, acc.at[idxd], add=True)
            return carry

        lax.fori_loop(0, n_groups, step, 0)
        plsc.subcore_barrier()

        @pl.when(s < _SC_NS - 1)
        def _():
            pltpu.sync_copy(acc.at[pl.ds(s * stripe, stripe)],
                            out_hbm.at[c, pl.ds(s * stripe, stripe)])

        @pl.when(s == _SC_NS - 1)
        def _():
            pltpu.sync_copy(acc.at[pl.ds(s * stripe, last)],
                            out_hbm.at[c, pl.ds(s * stripe, last)])

    f = pl.kernel(
        body,
        out_type=jax.ShapeDtypeStruct((_SC_NC, n, emb), jnp.float32),
        mesh=_sc_mesh(),
        scratch_types=[
            pltpu.VMEM_SHARED((n, emb), jnp.float32),
            pltpu.VMEM((_SC_G,), jnp.int32),
            pltpu.VMEM((_SC_G,), jnp.int32),
            pltpu.VMEM((_SC_G, emb), jnp.float32),
        ],
    )
    return f(h, src, dst, z2)


def _add_split_body(a_ref, b_ref, o0, o1, o2, o3):
    y = a_ref[...] + b_ref[...]
    for q, o in enumerate((o0, o1, o2, o3)):
        o[...] = y[:, 32 * q:32 * (q + 1)]


def _add_split(a, b):
    m, n = a.shape
    grid = m // _BM
    return pl.pallas_call(
        _add_split_body,
        grid=(grid,),
        in_specs=[pl.BlockSpec((_BM, n), lambda i: (i, 0))] * 2,
        out_specs=[pl.BlockSpec((_BM, 32), lambda i: (i, 0))] * 4,
        out_shape=[jax.ShapeDtypeStruct((m, 32), jnp.float32)] * 4,
    )(a, b)


def _mm_relu_slabs_body(s0, s1, s2, s3, w0, w1, w2, w3, b_ref, o_ref):
    y = b_ref[...]
    for s_ref, w_ref in ((s0, w0), (s1, w1), (s2, w2), (s3, w3)):
        y = y + jnp.dot(s_ref[...], w_ref[...],
                        preferred_element_type=jnp.float32)
    o_ref[...] = jnp.maximum(y, 0.0)


def _mm_relu_slabs(slabs, w, b):
    m = slabs[0].shape[0]
    n = w.shape[1]
    w_slices = [w[32 * q:32 * (q + 1), :] for q in range(4)]
    grid = m // _BM
    return pl.pallas_call(
        _mm_relu_slabs_body,
        grid=(grid,),
        in_specs=[pl.BlockSpec((_BM, 32), lambda i: (i, 0))] * 4
        + [pl.BlockSpec((32, n), lambda i: (0, 0))] * 4
        + [pl.BlockSpec((n,), lambda i: (0,))],
        out_specs=pl.BlockSpec((_BM, n), lambda i: (i, 0)),
        out_shape=jax.ShapeDtypeStruct((m, n), jnp.float32),
    )(*slabs, *w_slices, b)


def _sc_gather_xs(h, lab, nodeidx, nodelabel):
    n, emb = h.shape
    nsub = nodeidx.shape[0]
    n_groups = nsub // _SC_G
    k_max = (n_groups + _SC_NC * _SC_NS - 1) // (_SC_NC * _SC_NS)

    def body(h_hbm, lab_hbm, ni_hbm, nl_hbm, xh_hbm, xl_hbm,
             idx1, idx2, vals, vals2):
        c = lax.axis_index("c")
        s = lax.axis_index("s")
        wid = c * _SC_NS + s

        def step(k, carry):
            g = wid * k_max + k

            @pl.when(g < n_groups)
            def _():
                base = g * _SC_G
                pltpu.sync_copy(ni_hbm.at[pl.ds(base, _SC_G)], idx1)
                pltpu.sync_copy(nl_hbm.at[pl.ds(base, _SC_G)], idx2)
                pltpu.sync_copy(h_hbm.at[idx1], vals)
                pltpu.sync_copy(vals, xh_hbm.at[pl.ds(base, _SC_G)])
                pltpu.sync_copy(lab_hbm.at[idx2], vals2)
                pltpu.sync_copy(vals2, xl_hbm.at[pl.ds(base, _SC_G)])

            return carry

        lax.fori_loop(0, k_max, step, 0)

    f = pl.kernel(
        body,
        out_type=(jax.ShapeDtypeStruct((nsub, emb), jnp.float32),
                  jax.ShapeDtypeStruct((nsub, emb), jnp.float32)),
        mesh=_sc_mesh(),
        scratch_types=[
            pltpu.VMEM((_SC_G,), jnp.int32),
            pltpu.VMEM((_SC_G,), jnp.int32),
            pltpu.VMEM((_SC_G, emb), jnp.float32),
            pltpu.VMEM((_SC_G, emb), jnp.float32),
        ],
    )
    return f(h, lab, nodeidx, nodelabel)


def _sc_segsum_sub(xs_slabs, src, dst, zeros):
    nsub = xs_slabs[0].shape[0]
    e = src.shape[0]
    e_per_s = e // _SC_NS
    n_groups = e_per_s // _SC_G
    stripe = ((nsub // _SC_NS + 7) // 8) * 8
    last = nsub - stripe * (_SC_NS - 1)
    z2 = zeros[: stripe * 32].reshape(stripe, 32)

    def body(x0, x1, x2, x3, src_hbm, dst_hbm, z_hbm, a0, a1, a2, a3,
             acc, idxs, idxd, vals):
        c = lax.axis_index("c")
        s = lax.axis_index("s")
        xs_t = (x0, x1, x2, x3)
        ag_t = (a0, a1, a2, a3)
        for qi in range(2):
            for cc in range(_SC_NC):
                @pl.when(c == cc)
                def _(qi=qi, cc=cc):
                    xq = xs_t[2 * cc + qi]
                    aq = ag_t[2 * cc + qi]

                    @pl.when(s < _SC_NS - 1)
                    def _():
                        pltpu.sync_copy(z_hbm,
                                        acc.at[pl.ds(s * stripe, stripe)])

                    @pl.when(s == _SC_NS - 1)
                    def _():
                        pltpu.sync_copy(z_hbm.at[pl.ds(0, last)],
                                        acc.at[pl.ds(s * stripe, last)])

                    plsc.subcore_barrier()

                    def step(k, carry):
                        base = s * e_per_s + k * _SC_G
                        pltpu.sync_copy(src_hbm.at[pl.ds(base, _SC_G)], idxs)
                        pltpu.sync_copy(dst_hbm.at[pl.ds(base, _SC_G)], idxd)
                        pltpu.sync_copy(xq.at[idxs], vals)
                        pltpu.sync_copy(vals, acc.at[idxd], add=True)
                        return carry

                    lax.fori_loop(0, n_groups, step, 0)
                    plsc.subcore_barrier()

                    @pl.when(s < _SC_NS - 1)
                    def _():
                        pltpu.sync_copy(acc.at[pl.ds(s * stripe, stripe)],
                                        aq.at[pl.ds(s * stripe, stripe)])

                    @pl.when(s == _SC_NS - 1)
                    def _():
                        pltpu.sync_copy(acc.at[pl.ds(s * stripe, last)],
                                        aq.at[pl.ds(s * stripe, last)])

                    plsc.subcore_barrier()

    f = pl.kernel(
        body,
        out_type=tuple(jax.ShapeDtypeStruct((nsub, 32), jnp.float32)
                       for _ in range(4)),
        mesh=_sc_mesh(),
        compiler_params=pltpu.CompilerParams(use_tc_tiling_on_sc=False),
        scratch_types=[
            pltpu.VMEM_SHARED((nsub, 32), jnp.float32),
            pltpu.VMEM((_SC_G,), jnp.int32),
            pltpu.VMEM((_SC_G,), jnp.int32),
            pltpu.VMEM((_SC_G, 32), jnp.float32),
        ],
    )
    return f(*xs_slabs, src, dst, z2)


def _sc_scatter_mean(h_sub, nodeidx, zeros, n_out):
    nsub, emb = h_sub.shape
    n_groups = nsub // _SC_G
    k_max = (n_groups + _SC_NC * _SC_NS - 1) // (_SC_NC * _SC_NS)
    stripe = ((n_out // _SC_NS + 7) // 8) * 8
    last = n_out - stripe * (_SC_NS - 1)
    z_s = zeros[: stripe * emb].reshape(stripe, emb)
    ones = jnp.ones((_SC_G, emb), jnp.float32)

    def body(hs_hbm, ni_hbm, z_hbm, ones_hbm, sum_hbm, cnt_hbm,
             acc, idxd, vals, ones_v):
        c = lax.axis_index("c")
        s = lax.axis_index("s")
        wid = c * _SC_NS + s
        pltpu.sync_copy(ones_hbm, ones_v)

        def zero_acc():
            @pl.when(s < _SC_NS - 1)
            def _():
                pltpu.sync_copy(z_hbm, acc.at[pl.ds(s * stripe, stripe)])

            @pl.when(s == _SC_NS - 1)
            def _():
                pltpu.sync_copy(z_hbm.at[pl.ds(0, last)],
                                acc.at[pl.ds(s * stripe, last)])

        def flush_acc(dst_hbm):
            @pl.when(s < _SC_NS - 1)
            def _():
                pltpu.sync_copy(acc.at[pl.ds(s * stripe, stripe)],
                                dst_hbm.at[c, pl.ds(s * stripe, stripe)])

            @pl.when(s == _SC_NS - 1)
            def _():
                pltpu.sync_copy(acc.at[pl.ds(s * stripe, last)],
                                dst_hbm.at[c, pl.ds(s * stripe, last)])

        def scan_groups(do_group):
            def step(k, carry):
                g = wid * k_max + k

                @pl.when(g < n_groups)
                def _():
                    do_group(g * _SC_G)

                return carry

            lax.fori_loop(0, k_max, step, 0)

        zero_acc()
        plsc.subcore_barrier()

        def sum_group(base):
            pltpu.sync_copy(ni_hbm.at[pl.ds(base, _SC_G)], idxd)
            pltpu.sync_copy(hs_hbm.at[pl.ds(base, _SC_G)], vals)
            pltpu.sync_copy(vals, acc.at[idxd], add=True)

        scan_groups(sum_group)
        plsc.subcore_barrier()
        flush_acc(sum_hbm)
        plsc.subcore_barrier()

        zero_acc()
        plsc.subcore_barrier()

        def cnt_group(base):
            pltpu.sync_copy(ni_hbm.at[pl.ds(base, _SC_G)], idxd)
            pltpu.sync_copy(ones_v, acc.at[idxd], add=True)

        scan_groups(cnt_group)
        plsc.subcore_barrier()
        flush_acc(cnt_hbm)

    f = pl.kernel(
        body,
        out_type=(jax.ShapeDtypeStruct((_SC_NC, n_out, emb), jnp.float32),
                  jax.ShapeDtypeStruct((_SC_NC, n_out, emb), jnp.float32)),
        mesh=_sc_mesh(),
        scratch_types=[
            pltpu.VMEM_SHARED((n_out, emb), jnp.float32),
            pltpu.VMEM((_SC_G,), jnp.int32),
            pltpu.VMEM((_SC_G, emb), jnp.float32),
            pltpu.VMEM((_SC_G, emb), jnp.float32),
        ],
    )
    return f(h_sub, nodeidx, z_s, ones)


def kernel(x, edge_index, subg_nodeidx, subg_nodelabel, subg_edge_index, batch,
           W_enc, b_enc, label_emb, W_l, b_l, W_g, b_g, W_out, b_out):
    zeros = jnp.zeros((81920,), jnp.float32)

    h = _mm_relu([x], W_enc, b_enc)

    xh, xl = _sc_gather_xs(h, label_emb, subg_nodeidx, subg_nodelabel)
    xs_slabs = _add_split(xh, xl)

    agg_slabs = _sc_segsum_sub(xs_slabs, subg_edge_index[0],
                               subg_edge_index[1], zeros)

    h_sub = _mm_relu_slabs(agg_slabs, W_l, b_l)

    sums, cnts = _sc_scatter_mean(h_sub, subg_nodeidx, zeros, _N)
    h1 = _mean_residual(h, sums[0], sums[1], cnts[0], cnts[1])

    gp = _sc_segsum_edges(h1, edge_index[0], edge_index[1], zeros)

    h2 = _mm_relu([gp[0], gp[1]], W_g, b_g, residual=h1)

    return _pool_predict(h2, batch, W_out, b_out)


# speedup vs baseline: 3.5613x; 1.4007x over previous
import functools

import jax
import jax.numpy as jnp
from jax import lax
from jax.experimental import pallas as pl
from jax.experimental.pallas import tpu as pltpu
from jax.experimental.pallas import tpu_sc as plsc

_N = 10000
_E = 320000
_NSUB = 40000
_ESUB = 320000
_NGRAPH = 64
_EMB = 128

_BM = 2000


def _mm_relu_body(n_parts, residual, *refs):
    *x_refs, w_ref, b_ref, o_ref = refs
    acc = x_refs[0][...]
    for r in x_refs[1:n_parts]:
        acc = acc + r[...]
    y = jnp.dot(acc, w_ref[...], preferred_element_type=jnp.float32)
    y = jnp.maximum(y + b_ref[...], 0.0)
    if residual:
        y = x_refs[n_parts][...] + y
    o_ref[...] = y


def _mm_relu(parts, w, b, residual=None):
    m = parts[0].shape[0]
    k = parts[0].shape[1]
    n = w.shape[1]
    inputs = list(parts) + ([residual] if residual is not None else [])
    grid = m // _BM
    body = functools.partial(_mm_relu_body, len(parts), residual is not None)
    return pl.pallas_call(
        body,
        grid=(grid,),
        in_specs=[pl.BlockSpec((_BM, k), lambda i: (i, 0)) for _ in inputs]
        + [
            pl.BlockSpec((k, n), lambda i: (0, 0)),
            pl.BlockSpec((n,), lambda i: (0,)),
        ],
        out_specs=pl.BlockSpec((_BM, n), lambda i: (i, 0)),
        out_shape=jax.ShapeDtypeStruct((m, n), jnp.float32),
    )(*inputs, w, b)


def _mean_residual_body(h_ref, s0_ref, s1_ref, c0_ref, c1_ref, o_ref):
    cnt = jnp.maximum((c0_ref[...] + c1_ref[...])[:, 0:1], 1.0)
    o_ref[...] = h_ref[...] + (s0_ref[...] + s1_ref[...]) / cnt


def _mean_residual(h, s0, s1, c0, c1):
    m, n = h.shape
    grid = m // _BM
    return pl.pallas_call(
        _mean_residual_body,
        grid=(grid,),
        in_specs=[pl.BlockSpec((_BM, n), lambda i: (i, 0)) for _ in range(5)],
        out_specs=pl.BlockSpec((_BM, n), lambda i: (i, 0)),
        out_shape=jax.ShapeDtypeStruct((m, n), jnp.float32),
    )(h, s0, s1, c0, c1)


def _pool_body(h_ref, batch_ref, w_ref, b_ref, o_ref, acc_ref, cnt_ref):
    i = pl.program_id(0)

    @pl.when(i == 0)
    def _():
        acc_ref[...] = jnp.zeros_like(acc_ref)
        cnt_ref[...] = jnp.zeros_like(cnt_ref)

    seg = batch_ref[...].reshape(1, _BM)
    gids = lax.broadcasted_iota(jnp.int32, (_NGRAPH, _BM), 0)
    mask = (gids == seg).astype(jnp.float32)
    acc_ref[...] += jnp.dot(mask, h_ref[...], preferred_element_type=jnp.float32)
    cnt_ref[...] += jnp.sum(mask, axis=1, keepdims=True)

    @pl.when(i == pl.num_programs(0) - 1)
    def _():
        hg = acc_ref[...] / jnp.maximum(cnt_ref[...], 1.0)
        o_ref[...] = jnp.dot(hg, w_ref[...], preferred_element_type=jnp.float32) + b_ref[...]


def _pool_predict(h, batch, w_out, b_out):
    m, n = h.shape
    ntask = w_out.shape[1]
    w_pad = jnp.zeros((n, 128), jnp.float32).at[:, :ntask].set(w_out)
    b_pad = jnp.zeros((128,), jnp.float32).at[:ntask].set(b_out)
    batch3 = batch.reshape(m // _BM, 1, _BM)
    grid = m // _BM
    out = pl.pallas_call(
        _pool_body,
        grid=(grid,),
        in_specs=[
            pl.BlockSpec((_BM, n), lambda i: (i, 0)),
            pl.BlockSpec((1, 1, _BM), lambda i: (i, 0, 0)),
            pl.BlockSpec((n, 128), lambda i: (0, 0)),
            pl.BlockSpec((128,), lambda i: (0,)),
        ],
        out_specs=pl.BlockSpec((_NGRAPH, 128), lambda i: (0, 0)),
        out_shape=jax.ShapeDtypeStruct((_NGRAPH, 128), jnp.float32),
        scratch_shapes=[
            pltpu.VMEM((_NGRAPH, n), jnp.float32),
            pltpu.VMEM((_NGRAPH, 1), jnp.float32),
        ],
    )(h, batch3, w_pad, b_pad)
    return out[:, :ntask]


_SC_NC = 2
_SC_NS = 16
_SC_G = 80


def _sc_mesh():
    return plsc.VectorSubcoreMesh(core_axis_name="c", subcore_axis_name="s")


def _sc_segsum_edges(h, src, dst, zeros):
    n, emb = h.shape
    e = src.shape[0]
    e_per_w = e // (_SC_NC * _SC_NS)
    n_groups = e_per_w // _SC_G
    stripe = ((n // _SC_NS + 7) // 8) * 8
    last = n - stripe * (_SC_NS - 1)
    z2 = zeros[: stripe * emb].reshape(stripe, emb)

    def body(h_hbm, src_hbm, dst_hbm, z_hbm, out_hbm, acc, idxs, idxd, vals,
             sems):
        c = lax.axis_index("c")
        s = lax.axis_index("s")
        wid = c * _SC_NS + s

        @pl.when(s < _SC_NS - 1)
        def _():
            pltpu.sync_copy(z_hbm, acc.at[pl.ds(s * stripe, stripe)])

        @pl.when(s == _SC_NS - 1)
        def _():
            pltpu.sync_copy(z_hbm.at[pl.ds(0, last)],
                            acc.at[pl.ds(s * stripe, last)])

        plsc.subcore_barrier()
        base0 = wid * e_per_w
        _pipe_gather_scatter(h_hbm, src_hbm, dst_hbm,
                             lambda k: base0 + k * _SC_G, n_groups,
                             idxs, idxd, vals, sems, acc)
        plsc.subcore_barrier()

        @pl.when(s < _SC_NS - 1)
        def _():
            pltpu.sync_copy(acc.at[pl.ds(s * stripe, stripe)],
                            out_hbm.at[c, pl.ds(s * stripe, stripe)])

        @pl.when(s == _SC_NS - 1)
        def _():
            pltpu.sync_copy(acc.at[pl.ds(s * stripe, last)],
                            out_hbm.at[c, pl.ds(s * stripe, last)])

    f = pl.kernel(
        body,
        out_type=jax.ShapeDtypeStruct((_SC_NC, n, emb), jnp.float32),
        mesh=_sc_mesh(),
        scratch_types=[
            pltpu.VMEM_SHARED((n, emb), jnp.float32),
            pltpu.VMEM((2, _SC_G), jnp.int32),
            pltpu.VMEM((2, _SC_G), jnp.int32),
            pltpu.VMEM((2 * _SC_G, emb), jnp.float32),
            pltpu.SemaphoreType.DMA((2,)),
        ],
    )
    return f(h, src, dst, z2)


def _add_split_body(a_ref, b_ref, o0, o1, o2, o3):
    y = a_ref[...] + b_ref[...]
    for q, o in enumerate((o0, o1, o2, o3)):
        o[...] = y[:, 32 * q:32 * (q + 1)]


def _add_split(a, b):
    m, n = a.shape
    grid = m // _BM
    return pl.pallas_call(
        _add_split_body,
        grid=(grid,),
        in_specs=[pl.BlockSpec((_BM, n), lambda i: (i, 0))] * 2,
        out_specs=[pl.BlockSpec((_BM, 32), lambda i: (i, 0))] * 4,
        out_shape=[jax.ShapeDtypeStruct((m, 32), jnp.float32)] * 4,
    )(a, b)


def _mm_relu_slabs_body(s0, s1, s2, s3, w0, w1, w2, w3, b_ref, o_ref):
    y = b_ref[...]
    for s_ref, w_ref in ((s0, w0), (s1, w1), (s2, w2), (s3, w3)):
        y = y + jnp.dot(s_ref[...], w_ref[...],
                        preferred_element_type=jnp.float32)
    o_ref[...] = jnp.maximum(y, 0.0)


def _mm_relu_slabs(slabs, w, b):
    m = slabs[0].shape[0]
    n = w.shape[1]
    w_slices = [w[32 * q:32 * (q + 1), :] for q in range(4)]
    grid = m // _BM
    return pl.pallas_call(
        _mm_relu_slabs_body,
        grid=(grid,),
        in_specs=[pl.BlockSpec((_BM, 32), lambda i: (i, 0))] * 4
        + [pl.BlockSpec((32, n), lambda i: (0, 0))] * 4
        + [pl.BlockSpec((n,), lambda i: (0,))],
        out_specs=pl.BlockSpec((_BM, n), lambda i: (i, 0)),
        out_shape=jax.ShapeDtypeStruct((m, n), jnp.float32),
    )(*slabs, *w_slices, b)


def _pipe_gather_scatter(table, src_hbm, dst_hbm, base_fn, n_groups,
                         idxs2, idxd2, vals2, sems, acc):
    g = _SC_G

    def load_idx(k, slot):
        base = base_fn(k)
        pltpu.sync_copy(src_hbm.at[pl.ds(base, g)], idxs2.at[slot])
        pltpu.sync_copy(dst_hbm.at[pl.ds(base, g)], idxd2.at[slot])

    def gather_start(slot):
        pltpu.async_copy(table.at[idxs2.at[slot]],
                         vals2.at[pl.ds(slot * g, g)], sems.at[slot])

    def gather_wait(slot):
        pltpu.make_async_copy(table.at[idxs2.at[slot]],
                              vals2.at[pl.ds(slot * g, g)],
                              sems.at[slot]).wait()

    load_idx(0, 0)
    gather_start(0)

    def step(k, carry):
        slot = lax.rem(k, 2)
        nslot = 1 - slot

        @pl.when(k + 1 < n_groups)
        def _():
            load_idx(k + 1, nslot)
            gather_start(nslot)

        gather_wait(slot)
        pltpu.sync_copy(vals2.at[pl.ds(slot * g, g)], acc.at[idxd2.at[slot]],
                        add=True)
        return carry

    lax.fori_loop(0, n_groups, step, 0)


def _sc_gather_xs(h, lab, nodeidx, nodelabel):
    n, emb = h.shape
    nsub = nodeidx.shape[0]
    n_groups = nsub // _SC_G
    k_max = (n_groups + _SC_NC * _SC_NS - 1) // (_SC_NC * _SC_NS)

    def body(h_hbm, lab_hbm, ni_hbm, nl_hbm, xh_hbm, xl_hbm,
             idx1, idx2, vals, vals2):
        c = lax.axis_index("c")
        s = lax.axis_index("s")
        wid = c * _SC_NS + s

        def step(k, carry):
            g = wid * k_max + k

            @pl.when(g < n_groups)
            def _():
                base = g * _SC_G
                pltpu.sync_copy(ni_hbm.at[pl.ds(base, _SC_G)], idx1)
                pltpu.sync_copy(nl_hbm.at[pl.ds(base, _SC_G)], idx2)
                pltpu.sync_copy(h_hbm.at[idx1], vals)
                pltpu.sync_copy(vals, xh_hbm.at[pl.ds(base, _SC_G)])
                pltpu.sync_copy(lab_hbm.at[idx2], vals2)
                pltpu.sync_copy(vals2, xl_hbm.at[pl.ds(base, _SC_G)])

            return carry

        lax.fori_loop(0, k_max, step, 0)

    f = pl.kernel(
        body,
        out_type=(jax.ShapeDtypeStruct((nsub, emb), jnp.float32),
                  jax.ShapeDtypeStruct((nsub, emb), jnp.float32)),
        mesh=_sc_mesh(),
        scratch_types=[
            pltpu.VMEM((_SC_G,), jnp.int32),
            pltpu.VMEM((_SC_G,), jnp.int32),
            pltpu.VMEM((_SC_G, emb), jnp.float32),
            pltpu.VMEM((_SC_G, emb), jnp.float32),
        ],
    )
    return f(h, lab, nodeidx, nodelabel)


def _sc_segsum_sub(xs_slabs, src, dst, zeros):
    nsub = xs_slabs[0].shape[0]
    e = src.shape[0]
    e_per_s = e // _SC_NS
    n_groups = e_per_s // _SC_G
    stripe = ((nsub // _SC_NS + 7) // 8) * 8
    last = nsub - stripe * (_SC_NS - 1)
    z2 = zeros[: stripe * 32].reshape(stripe, 32)

    def body(x0, x1, x2, x3, src_hbm, dst_hbm, z_hbm, a0, a1, a2, a3,
             acc, idxs, idxd, vals, sems):
        c = lax.axis_index("c")
        s = lax.axis_index("s")
        xs_t = (x0, x1, x2, x3)
        ag_t = (a0, a1, a2, a3)
        for qi in range(2):
            for cc in range(_SC_NC):
                @pl.when(c == cc)
                def _(qi=qi, cc=cc):
                    xq = xs_t[2 * cc + qi]
                    aq = ag_t[2 * cc + qi]

                    @pl.when(s < _SC_NS - 1)
                    def _():
                        pltpu.sync_copy(z_hbm,
                                        acc.at[pl.ds(s * stripe, stripe)])

                    @pl.when(s == _SC_NS - 1)
                    def _():
                        pltpu.sync_copy(z_hbm.at[pl.ds(0, last)],
                                        acc.at[pl.ds(s * stripe, last)])

                    plsc.subcore_barrier()
                    _pipe_gather_scatter(
                        xq, src_hbm, dst_hbm,
                        lambda k: s * e_per_s + k * _SC_G, n_groups,
                        idxs, idxd, vals, sems, acc)
                    plsc.subcore_barrier()

                    @pl.when(s < _SC_NS - 1)
                    def _():
                        pltpu.sync_copy(acc.at[pl.ds(s * stripe, stripe)],
                                        aq.at[pl.ds(s * stripe, stripe)])

                    @pl.when(s == _SC_NS - 1)
                    def _():
                        pltpu.sync_copy(acc.at[pl.ds(s * stripe, last)],
                                        aq.at[pl.ds(s * stripe, last)])

                    plsc.subcore_barrier()

    f = pl.kernel(
        body,
        out_type=tuple(jax.ShapeDtypeStruct((nsub, 32), jnp.float32)
                       for _ in range(4)),
        mesh=_sc_mesh(),
        compiler_params=pltpu.CompilerParams(use_tc_tiling_on_sc=False),
        scratch_types=[
            pltpu.VMEM_SHARED((nsub, 32), jnp.float32),
            pltpu.VMEM((2, _SC_G), jnp.int32),
            pltpu.VMEM((2, _SC_G), jnp.int32),
            pltpu.VMEM((2 * _SC_G, 32), jnp.float32),
            pltpu.SemaphoreType.DMA((2,)),
        ],
    )
    return f(*xs_slabs, src, dst, z2)


def _sc_scatter_mean(h_sub, nodeidx, zeros, n_out):
    nsub, emb = h_sub.shape
    n_groups = nsub // _SC_G
    k_max = (n_groups + _SC_NC * _SC_NS - 1) // (_SC_NC * _SC_NS)
    stripe = ((n_out // _SC_NS + 7) // 8) * 8
    last = n_out - stripe * (_SC_NS - 1)
    z_s = zeros[: stripe * emb].reshape(stripe, emb)
    ones = jnp.ones((_SC_G, emb), jnp.float32)

    def body(hs_hbm, ni_hbm, z_hbm, ones_hbm, sum_hbm, cnt_hbm,
             acc, idxd, vals, ones_v):
        c = lax.axis_index("c")
        s = lax.axis_index("s")
        wid = c * _SC_NS + s
        pltpu.sync_copy(ones_hbm, ones_v)

        def zero_acc():
            @pl.when(s < _SC_NS - 1)
            def _():
                pltpu.sync_copy(z_hbm, acc.at[pl.ds(s * stripe, stripe)])

            @pl.when(s == _SC_NS - 1)
            def _():
                pltpu.sync_copy(z_hbm.at[pl.ds(0, last)],
                                acc.at[pl.ds(s * stripe, last)])

        def flush_acc(dst_hbm):
            @pl.when(s < _SC_NS - 1)
            def _():
                pltpu.sync_copy(acc.at[pl.ds(s * stripe, stripe)],
                                dst_hbm.at[c, pl.ds(s * stripe, stripe)])

            @pl.when(s == _SC_NS - 1)
            def _():
                pltpu.sync_copy(acc.at[pl.ds(s * stripe, last)],
                                dst_hbm.at[c, pl.ds(s * stripe, last)])

        def scan_groups(do_group):
            def step(k, carry):
                g = wid * k_max + k

                @pl.when(g < n_groups)
                def _():
                    do_group(g * _SC_G)

                return carry

            lax.fori_loop(0, k_max, step, 0)

        zero_acc()
        plsc.subcore_barrier()

        def sum_group(base):
            pltpu.sync_copy(ni_hbm.at[pl.ds(base, _SC_G)], idxd)
            pltpu.sync_copy(hs_hbm.at[pl.ds(base, _SC_G)], vals)
            pltpu.sync_copy(vals, acc.at[idxd], add=True)

        scan_groups(sum_group)
        plsc.subcore_barrier()
        flush_acc(sum_hbm)
        plsc.subcore_barrier()

        zero_acc()
        plsc.subcore_barrier()

        def cnt_group(base):
            pltpu.sync_copy(ni_hbm.at[pl.ds(base, _SC_G)], idxd)
            pltpu.sync_copy(ones_v, acc.at[idxd], add=True)

        scan_groups(cnt_group)
        plsc.subcore_barrier()
        flush_acc(cnt_hbm)

    f = pl.kernel(
        body,
        out_type=(jax.ShapeDtypeStruct((_SC_NC, n_out, emb), jnp.float32),
                  jax.ShapeDtypeStruct((_SC_NC, n_out, emb), jnp.float32)),
        mesh=_sc_mesh(),
        scratch_types=[
            pltpu.VMEM_SHARED((n_out, emb), jnp.float32),
            pltpu.VMEM((_SC_G,), jnp.int32),
            pltpu.VMEM((_SC_G, emb), jnp.float32),
            pltpu.VMEM((_SC_G, emb), jnp.float32),
        ],
    )
    return f(h_sub, nodeidx, z_s, ones)


def kernel(x, edge_index, subg_nodeidx, subg_nodelabel, subg_edge_index, batch,
           W_enc, b_enc, label_emb, W_l, b_l, W_g, b_g, W_out, b_out):
    zeros = jnp.zeros((81920,), jnp.float32)

    h = _mm_relu([x], W_enc, b_enc)

    xh, xl = _sc_gather_xs(h, label_emb, subg_nodeidx, subg_nodelabel)
    xs_slabs = _add_split(xh, xl)

    agg_slabs = _sc_segsum_sub(xs_slabs, subg_edge_index[0],
                               subg_edge_index[1], zeros)

    h_sub = _mm_relu_slabs(agg_slabs, W_l, b_l)

    sums, cnts = _sc_scatter_mean(h_sub, subg_nodeidx, zeros, _N)
    h1 = _mean_residual(h, sums[0], sums[1], cnts[0], cnts[1])

    gp = _sc_segsum_edges(h1, edge_index[0], edge_index[1], zeros)

    h2 = _mm_relu([gp[0], gp[1]], W_g, b_g, residual=h1)

    return _pool_predict(h2, batch, W_out, b_out)


# speedup vs baseline: 4.8970x; 1.3751x over previous
import functools

import jax
import jax.numpy as jnp
from jax import lax
from jax.experimental import pallas as pl
from jax.experimental.pallas import tpu as pltpu
from jax.experimental.pallas import tpu_sc as plsc

_N = 10000
_E = 320000
_NSUB = 40000
_ESUB = 320000
_NGRAPH = 64
_EMB = 128

_BM = 2000


def _mm_relu_body(n_parts, residual, *refs):
    *x_refs, w_ref, b_ref, o_ref = refs
    acc = x_refs[0][...]
    for r in x_refs[1:n_parts]:
        acc = acc + r[...]
    y = jnp.dot(acc, w_ref[...], preferred_element_type=jnp.float32)
    y = jnp.maximum(y + b_ref[...], 0.0)
    if residual:
        y = x_refs[n_parts][...] + y
    o_ref[...] = y


def _mm_relu(parts, w, b, residual=None):
    m = parts[0].shape[0]
    k = parts[0].shape[1]
    n = w.shape[1]
    inputs = list(parts) + ([residual] if residual is not None else [])
    grid = m // _BM
    body = functools.partial(_mm_relu_body, len(parts), residual is not None)
    return pl.pallas_call(
        body,
        grid=(grid,),
        in_specs=[pl.BlockSpec((_BM, k), lambda i: (i, 0)) for _ in inputs]
        + [
            pl.BlockSpec((k, n), lambda i: (0, 0)),
            pl.BlockSpec((n,), lambda i: (0,)),
        ],
        out_specs=pl.BlockSpec((_BM, n), lambda i: (i, 0)),
        out_shape=jax.ShapeDtypeStruct((m, n), jnp.float32),
    )(*inputs, w, b)


def _mean_residual_body(h_ref, s0_ref, s1_ref, c0_ref, c1_ref, o_ref):
    cnt = jnp.maximum((c0_ref[...] + c1_ref[...])[:, 0:1], 1.0)
    o_ref[...] = h_ref[...] + (s0_ref[...] + s1_ref[...]) / cnt


def _mean_residual(h, s0, s1, c0, c1):
    m, n = h.shape
    grid = m // _BM
    return pl.pallas_call(
        _mean_residual_body,
        grid=(grid,),
        in_specs=[pl.BlockSpec((_BM, n), lambda i: (i, 0)) for _ in range(5)],
        out_specs=pl.BlockSpec((_BM, n), lambda i: (i, 0)),
        out_shape=jax.ShapeDtypeStruct((m, n), jnp.float32),
    )(h, s0, s1, c0, c1)


def _pool_body(h_ref, batch_ref, w_ref, b_ref, o_ref, acc_ref, cnt_ref):
    i = pl.program_id(0)

    @pl.when(i == 0)
    def _():
        acc_ref[...] = jnp.zeros_like(acc_ref)
        cnt_ref[...] = jnp.zeros_like(cnt_ref)

    seg = batch_ref[...].reshape(1, _BM)
    gids = lax.broadcasted_iota(jnp.int32, (_NGRAPH, _BM), 0)
    mask = (gids == seg).astype(jnp.float32)
    acc_ref[...] += jnp.dot(mask, h_ref[...], preferred_element_type=jnp.float32)
    cnt_ref[...] += jnp.sum(mask, axis=1, keepdims=True)

    @pl.when(i == pl.num_programs(0) - 1)
    def _():
        hg = acc_ref[...] / jnp.maximum(cnt_ref[...], 1.0)
        o_ref[...] = jnp.dot(hg, w_ref[...], preferred_element_type=jnp.float32) + b_ref[...]


def _pool_predict(h, batch, w_out, b_out):
    m, n = h.shape
    ntask = w_out.shape[1]
    w_pad = jnp.zeros((n, 128), jnp.float32).at[:, :ntask].set(w_out)
    b_pad = jnp.zeros((128,), jnp.float32).at[:ntask].set(b_out)
    batch3 = batch.reshape(m // _BM, 1, _BM)
    grid = m // _BM
    out = pl.pallas_call(
        _pool_body,
        grid=(grid,),
        in_specs=[
            pl.BlockSpec((_BM, n), lambda i: (i, 0)),
            pl.BlockSpec((1, 1, _BM), lambda i: (i, 0, 0)),
            pl.BlockSpec((n, 128), lambda i: (0, 0)),
            pl.BlockSpec((128,), lambda i: (0,)),
        ],
        out_specs=pl.BlockSpec((_NGRAPH, 128), lambda i: (0, 0)),
        out_shape=jax.ShapeDtypeStruct((_NGRAPH, 128), jnp.float32),
        scratch_shapes=[
            pltpu.VMEM((_NGRAPH, n), jnp.float32),
            pltpu.VMEM((_NGRAPH, 1), jnp.float32),
        ],
    )(h, batch3, w_pad, b_pad)
    return out[:, :ntask]


_SC_NC = 2
_SC_NS = 16
_SC_G = 80
_NB = 25


def _sc_mesh():
    return plsc.VectorSubcoreMesh(core_axis_name="c", subcore_axis_name="s")


def _sc_segsum_edges(h, src, dst, zeros):
    n, emb = h.shape
    e = src.shape[0]
    e_per_w = e // (_SC_NC * _SC_NS)
    n_groups = e_per_w // _SC_G
    stripe = ((n // _SC_NS + 7) // 8) * 8
    last = n - stripe * (_SC_NS - 1)
    z2 = zeros[: stripe * emb].reshape(stripe, emb)
    nw = _SC_NC * _SC_NS
    src3 = src.reshape(nw, n_groups, _SC_G)
    dst3 = dst.reshape(nw, n_groups, _SC_G)

    def body(h_hbm, src_hbm, dst_hbm, z_hbm, out_hbm, acc, idxs, idxd, vals,
             sems):
        c = lax.axis_index("c")
        s = lax.axis_index("s")
        wid = c * _SC_NS + s

        @pl.when(s < _SC_NS - 1)
        def _():
            pltpu.sync_copy(z_hbm, acc.at[pl.ds(s * stripe, stripe)])

        @pl.when(s == _SC_NS - 1)
        def _():
            pltpu.sync_copy(z_hbm.at[pl.ds(0, last)],
                            acc.at[pl.ds(s * stripe, last)])

        plsc.subcore_barrier()

        def block(b, carry):
            pltpu.sync_copy(src_hbm.at[wid, pl.ds(b * _NB, _NB)], idxs)
            pltpu.sync_copy(dst_hbm.at[wid, pl.ds(b * _NB, _NB)], idxd)
            _pipe_gather_scatter(h_hbm, idxs, idxd, _NB, vals, sems, acc)
            return carry

        lax.fori_loop(0, n_groups // _NB, block, 0)
        plsc.subcore_barrier()

        @pl.when(s < _SC_NS - 1)
        def _():
            pltpu.sync_copy(acc.at[pl.ds(s * stripe, stripe)],
                            out_hbm.at[c, pl.ds(s * stripe, stripe)])

        @pl.when(s == _SC_NS - 1)
        def _():
            pltpu.sync_copy(acc.at[pl.ds(s * stripe, last)],
                            out_hbm.at[c, pl.ds(s * stripe, last)])

    f = pl.kernel(
        body,
        out_type=jax.ShapeDtypeStruct((_SC_NC, n, emb), jnp.float32),
        mesh=_sc_mesh(),
        compiler_params=pltpu.CompilerParams(use_tc_tiling_on_sc=False),
        scratch_types=[
            pltpu.VMEM_SHARED((n, emb), jnp.float32),
            pltpu.VMEM((_NB, _SC_G), jnp.int32),
            pltpu.VMEM((_NB, _SC_G), jnp.int32),
            pltpu.VMEM((2 * _SC_G, emb), jnp.float32),
            pltpu.SemaphoreType.DMA((2,)),
        ],
    )
    return f(h, src3, dst3, z2)


def _add_split_body(a_ref, b_ref, o0, o1, o2, o3):
    y = a_ref[...] + b_ref[...]
    for q, o in enumerate((o0, o1, o2, o3)):
        o[...] = y[:, 32 * q:32 * (q + 1)]


def _add_split(a, b):
    m, n = a.shape
    grid = m // _BM
    return pl.pallas_call(
        _add_split_body,
        grid=(grid,),
        in_specs=[pl.BlockSpec((_BM, n), lambda i: (i, 0))] * 2,
        out_specs=[pl.BlockSpec((_BM, 32), lambda i: (i, 0))] * 4,
        out_shape=[jax.ShapeDtypeStruct((m, 32), jnp.float32)] * 4,
    )(a, b)


def _mm_relu_slabs_body(s0, s1, s2, s3, w0, w1, w2, w3, b_ref, o_ref):
    y = b_ref[...]
    for s_ref, w_ref in ((s0, w0), (s1, w1), (s2, w2), (s3, w3)):
        y = y + jnp.dot(s_ref[...], w_ref[...],
                        preferred_element_type=jnp.float32)
    o_ref[...] = jnp.maximum(y, 0.0)


def _mm_relu_slabs(slabs, w, b):
    m = slabs[0].shape[0]
    n = w.shape[1]
    w_slices = [w[32 * q:32 * (q + 1), :] for q in range(4)]
    grid = m // _BM
    return pl.pallas_call(
        _mm_relu_slabs_body,
        grid=(grid,),
        in_specs=[pl.BlockSpec((_BM, 32), lambda i: (i, 0))] * 4
        + [pl.BlockSpec((32, n), lambda i: (0, 0))] * 4
        + [pl.BlockSpec((n,), lambda i: (0,))],
        out_specs=pl.BlockSpec((_BM, n), lambda i: (i, 0)),
        out_shape=jax.ShapeDtypeStruct((m, n), jnp.float32),
    )(*slabs, *w_slices, b)


def _pipe_gather_scatter(table, idxs_v, idxd_v, n_groups, vals2, sems, acc):
    g = _SC_G

    def gather_start(k, slot):
        pltpu.async_copy(table.at[idxs_v.at[k]],
                         vals2.at[pl.ds(slot * g, g)], sems.at[slot])

    def gather_wait(k, slot):
        pltpu.make_async_copy(table.at[idxs_v.at[k]],
                              vals2.at[pl.ds(slot * g, g)],
                              sems.at[slot]).wait()

    gather_start(0, 0)

    def step(k, carry):
        slot = lax.rem(k, 2)

        @pl.when(k + 1 < n_groups)
        def _():
            gather_start(k + 1, 1 - slot)

        gather_wait(k, slot)
        pltpu.sync_copy(vals2.at[pl.ds(slot * g, g)], acc.at[idxd_v.at[k]],
                        add=True)
        return carry

    lax.fori_loop(0, n_groups, step, 0)


def _sc_gather_xs(h, lab, nodeidx, nodelabel):
    n, emb = h.shape
    nsub = nodeidx.shape[0]
    n_groups = nsub // _SC_G
    k_max = (n_groups + _SC_NC * _SC_NS - 1) // (_SC_NC * _SC_NS)

    def body(h_hbm, lab_hbm, ni_hbm, nl_hbm, xh_hbm, xl_hbm,
             idx1, idx2, vals, vals2):
        c = lax.axis_index("c")
        s = lax.axis_index("s")
        wid = c * _SC_NS + s

        def step(k, carry):
            g = wid * k_max + k

            @pl.when(g < n_groups)
            def _():
                base = g * _SC_G
                pltpu.sync_copy(ni_hbm.at[pl.ds(base, _SC_G)], idx1)
                pltpu.sync_copy(nl_hbm.at[pl.ds(base, _SC_G)], idx2)
                pltpu.sync_copy(h_hbm.at[idx1], vals)
                pltpu.sync_copy(vals, xh_hbm.at[pl.ds(base, _SC_G)])
                pltpu.sync_copy(lab_hbm.at[idx2], vals2)
                pltpu.sync_copy(vals2, xl_hbm.at[pl.ds(base, _SC_G)])

            return carry

        lax.fori_loop(0, k_max, step, 0)

    f = pl.kernel(
        body,
        out_type=(jax.ShapeDtypeStruct((nsub, emb), jnp.float32),
                  jax.ShapeDtypeStruct((nsub, emb), jnp.float32)),
        mesh=_sc_mesh(),
        scratch_types=[
            pltpu.VMEM((_SC_G,), jnp.int32),
            pltpu.VMEM((_SC_G,), jnp.int32),
            pltpu.VMEM((_SC_G, emb), jnp.float32),
            pltpu.VMEM((_SC_G, emb), jnp.float32),
        ],
    )
    return f(h, lab, nodeidx, nodelabel)


def _sc_segsum_sub(xs_slabs, src, dst, zeros):
    nsub = xs_slabs[0].shape[0]
    e = src.shape[0]
    e_per_s = e // _SC_NS
    n_groups = e_per_s // _SC_G
    stripe = ((nsub // _SC_NS + 7) // 8) * 8
    last = nsub - stripe * (_SC_NS - 1)
    z2 = zeros[: stripe * 32].reshape(stripe, 32)
    src3 = src.reshape(_SC_NS, n_groups, _SC_G)
    dst3 = dst.reshape(_SC_NS, n_groups, _SC_G)

    def body(x0, x1, x2, x3, src_hbm, dst_hbm, z_hbm, a0, a1, a2, a3,
             acc, idxs, idxd, vals, sems):
        c = lax.axis_index("c")
        s = lax.axis_index("s")
        xs_t = (x0, x1, x2, x3)
        ag_t = (a0, a1, a2, a3)
        for qi in range(2):
            for cc in range(_SC_NC):
                @pl.when(c == cc)
                def _(qi=qi, cc=cc):
                    xq = xs_t[2 * cc + qi]
                    aq = ag_t[2 * cc + qi]

                    @pl.when(s < _SC_NS - 1)
                    def _():
                        pltpu.sync_copy(z_hbm,
                                        acc.at[pl.ds(s * stripe, stripe)])

                    @pl.when(s == _SC_NS - 1)
                    def _():
                        pltpu.sync_copy(z_hbm.at[pl.ds(0, last)],
                                        acc.at[pl.ds(s * stripe, last)])

                    plsc.subcore_barrier()

                    def block(b, carry, xq=xq):
                        pltpu.sync_copy(src_hbm.at[s, pl.ds(b * _NB, _NB)],
                                        idxs)
                        pltpu.sync_copy(dst_hbm.at[s, pl.ds(b * _NB, _NB)],
                                        idxd)
                        _pipe_gather_scatter(xq, idxs, idxd, _NB,
                                             vals, sems, acc)
                        return carry

                    lax.fori_loop(0, n_groups // _NB, block, 0)
                    plsc.subcore_barrier()

                    @pl.when(s < _SC_NS - 1)
                    def _():
                        pltpu.sync_copy(acc.at[pl.ds(s * stripe, stripe)],
                                        aq.at[pl.ds(s * stripe, stripe)])

                    @pl.when(s == _SC_NS - 1)
                    def _():
                        pltpu.sync_copy(acc.at[pl.ds(s * stripe, last)],
                                        aq.at[pl.ds(s * stripe, last)])

                    plsc.subcore_barrier()

    f = pl.kernel(
        body,
        out_type=tuple(jax.ShapeDtypeStruct((nsub, 32), jnp.float32)
                       for _ in range(4)),
        mesh=_sc_mesh(),
        compiler_params=pltpu.CompilerParams(use_tc_tiling_on_sc=False),
        scratch_types=[
            pltpu.VMEM_SHARED((nsub, 32), jnp.float32),
            pltpu.VMEM((_NB, _SC_G), jnp.int32),
            pltpu.VMEM((_NB, _SC_G), jnp.int32),
            pltpu.VMEM((2 * _SC_G, 32), jnp.float32),
            pltpu.SemaphoreType.DMA((2,)),
        ],
    )
    return f(*xs_slabs, src3, dst3, z2)


def _sc_scatter_mean(h_sub, nodeidx, zeros, n_out):
    nsub, emb = h_sub.shape
    n_groups = nsub // _SC_G
    k_max = (n_groups + _SC_NC * _SC_NS - 1) // (_SC_NC * _SC_NS)
    stripe = ((n_out // _SC_NS + 7) // 8) * 8
    last = n_out - stripe * (_SC_NS - 1)
    z_s = zeros[: stripe * emb].reshape(stripe, emb)
    ones = jnp.ones((_SC_G, emb), jnp.float32)

    def body(hs_hbm, ni_hbm, z_hbm, ones_hbm, sum_hbm, cnt_hbm,
             acc, idxd, vals, ones_v):
        c = lax.axis_index("c")
        s = lax.axis_index("s")
        wid = c * _SC_NS + s
        pltpu.sync_copy(ones_hbm, ones_v)

        def zero_acc():
            @pl.when(s < _SC_NS - 1)
            def _():
                pltpu.sync_copy(z_hbm, acc.at[pl.ds(s * stripe, stripe)])

            @pl.when(s == _SC_NS - 1)
            def _():
                pltpu.sync_copy(z_hbm.at[pl.ds(0, last)],
                                acc.at[pl.ds(s * stripe, last)])

        def flush_acc(dst_hbm):
            @pl.when(s < _SC_NS - 1)
            def _():
                pltpu.sync_copy(acc.at[pl.ds(s * stripe, stripe)],
                                dst_hbm.at[c, pl.ds(s * stripe, stripe)])

            @pl.when(s == _SC_NS - 1)
            def _():
                pltpu.sync_copy(acc.at[pl.ds(s * stripe, last)],
                                dst_hbm.at[c, pl.ds(s * stripe, last)])

        def scan_groups(do_group):
            def step(k, carry):
                g = wid * k_max + k

                @pl.when(g < n_groups)
                def _():
                    do_group(g * _SC_G)

                return carry

            lax.fori_loop(0, k_max, step, 0)

        zero_acc()
        plsc.subcore_barrier()

        def sum_group(base):
            pltpu.sync_copy(ni_hbm.at[pl.ds(base, _SC_G)], idxd)
            pltpu.sync_copy(hs_hbm.at[pl.ds(base, _SC_G)], vals)
            pltpu.sync_copy(vals, acc.at[idxd], add=True)

        scan_groups(sum_group)
        plsc.subcore_barrier()
        flush_acc(sum_hbm)
        plsc.subcore_barrier()

        zero_acc()
        plsc.subcore_barrier()

        def cnt_group(base):
            pltpu.sync_copy(ni_hbm.at[pl.ds(base, _SC_G)], idxd)
            pltpu.sync_copy(ones_v, acc.at[idxd], add=True)

        scan_groups(cnt_group)
        plsc.subcore_barrier()
        flush_acc(cnt_hbm)

    f = pl.kernel(
        body,
        out_type=(jax.ShapeDtypeStruct((_SC_NC, n_out, emb), jnp.float32),
                  jax.ShapeDtypeStruct((_SC_NC, n_out, emb), jnp.float32)),
        mesh=_sc_mesh(),
        scratch_types=[
            pltpu.VMEM_SHARED((n_out, emb), jnp.float32),
            pltpu.VMEM((_SC_G,), jnp.int32),
            pltpu.VMEM((_SC_G, emb), jnp.float32),
            pltpu.VMEM((_SC_G, emb), jnp.float32),
        ],
    )
    return f(h_sub, nodeidx, z_s, ones)


def kernel(x, edge_index, subg_nodeidx, subg_nodelabel, subg_edge_index, batch,
           W_enc, b_enc, label_emb, W_l, b_l, W_g, b_g, W_out, b_out):
    zeros = jnp.zeros((81920,), jnp.float32)

    h = _mm_relu([x], W_enc, b_enc)

    xh, xl = _sc_gather_xs(h, label_emb, subg_nodeidx, subg_nodelabel)
    xs_slabs = _add_split(xh, xl)

    agg_slabs = _sc_segsum_sub(xs_slabs, subg_edge_index[0],
                               subg_edge_index[1], zeros)

    h_sub = _mm_relu_slabs(agg_slabs, W_l, b_l)

    sums, cnts = _sc_scatter_mean(h_sub, subg_nodeidx, zeros, _N)
    h1 = _mean_residual(h, sums[0], sums[1], cnts[0], cnts[1])

    gp = _sc_segsum_edges(h1, edge_index[0], edge_index[1], zeros)

    h2 = _mm_relu([gp[0], gp[1]], W_g, b_g, residual=h1)

    return _pool_predict(h2, batch, W_out, b_out)


# speedup vs baseline: 4.8994x; 1.0005x over previous
import functools

import jax
import jax.numpy as jnp
from jax import lax
from jax.experimental import pallas as pl
from jax.experimental.pallas import tpu as pltpu
from jax.experimental.pallas import tpu_sc as plsc

_N = 10000
_E = 320000
_NSUB = 40000
_ESUB = 320000
_NGRAPH = 64
_EMB = 128

_BM = 2000


def _mm_relu_body(n_parts, residual, *refs):
    *x_refs, w_ref, b_ref, o_ref = refs
    acc = x_refs[0][...]
    for r in x_refs[1:n_parts]:
        acc = acc + r[...]
    y = jnp.dot(acc, w_ref[...], preferred_element_type=jnp.float32)
    y = jnp.maximum(y + b_ref[...], 0.0)
    if residual:
        y = x_refs[n_parts][...] + y
    o_ref[...] = y


def _mm_relu(parts, w, b, residual=None):
    m = parts[0].shape[0]
    k = parts[0].shape[1]
    n = w.shape[1]
    inputs = list(parts) + ([residual] if residual is not None else [])
    grid = m // _BM
    body = functools.partial(_mm_relu_body, len(parts), residual is not None)
    return pl.pallas_call(
        body,
        grid=(grid,),
        in_specs=[pl.BlockSpec((_BM, k), lambda i: (i, 0)) for _ in inputs]
        + [
            pl.BlockSpec((k, n), lambda i: (0, 0)),
            pl.BlockSpec((n,), lambda i: (0,)),
        ],
        out_specs=pl.BlockSpec((_BM, n), lambda i: (i, 0)),
        out_shape=jax.ShapeDtypeStruct((m, n), jnp.float32),
    )(*inputs, w, b)


def _mean_residual_body(h_ref, s0_ref, s1_ref, c0_ref, c1_ref, o_ref):
    cnt = jnp.maximum((c0_ref[...] + c1_ref[...])[:, 0:1], 1.0)
    o_ref[...] = h_ref[...] + (s0_ref[...] + s1_ref[...]) / cnt


def _mean_residual(h, s0, s1, c0, c1):
    m, n = h.shape
    grid = m // _BM
    return pl.pallas_call(
        _mean_residual_body,
        grid=(grid,),
        in_specs=[pl.BlockSpec((_BM, n), lambda i: (i, 0)) for _ in range(5)],
        out_specs=pl.BlockSpec((_BM, n), lambda i: (i, 0)),
        out_shape=jax.ShapeDtypeStruct((m, n), jnp.float32),
    )(h, s0, s1, c0, c1)


def _pool_body(h_ref, batch_ref, w_ref, b_ref, o_ref, acc_ref, cnt_ref):
    i = pl.program_id(0)

    @pl.when(i == 0)
    def _():
        acc_ref[...] = jnp.zeros_like(acc_ref)
        cnt_ref[...] = jnp.zeros_like(cnt_ref)

    seg = batch_ref[...].reshape(1, _BM)
    gids = lax.broadcasted_iota(jnp.int32, (_NGRAPH, _BM), 0)
    mask = (gids == seg).astype(jnp.float32)
    acc_ref[...] += jnp.dot(mask, h_ref[...], preferred_element_type=jnp.float32)
    cnt_ref[...] += jnp.sum(mask, axis=1, keepdims=True)

    @pl.when(i == pl.num_programs(0) - 1)
    def _():
        hg = acc_ref[...] / jnp.maximum(cnt_ref[...], 1.0)
        o_ref[...] = jnp.dot(hg, w_ref[...], preferred_element_type=jnp.float32) + b_ref[...]


def _pool_predict(h, batch, w_out, b_out):
    m, n = h.shape
    ntask = w_out.shape[1]
    w_pad = jnp.zeros((n, 128), jnp.float32).at[:, :ntask].set(w_out)
    b_pad = jnp.zeros((128,), jnp.float32).at[:ntask].set(b_out)
    batch3 = batch.reshape(m // _BM, 1, _BM)
    grid = m // _BM
    out = pl.pallas_call(
        _pool_body,
        grid=(grid,),
        in_specs=[
            pl.BlockSpec((_BM, n), lambda i: (i, 0)),
            pl.BlockSpec((1, 1, _BM), lambda i: (i, 0, 0)),
            pl.BlockSpec((n, 128), lambda i: (0, 0)),
            pl.BlockSpec((128,), lambda i: (0,)),
        ],
        out_specs=pl.BlockSpec((_NGRAPH, 128), lambda i: (0, 0)),
        out_shape=jax.ShapeDtypeStruct((_NGRAPH, 128), jnp.float32),
        scratch_shapes=[
            pltpu.VMEM((_NGRAPH, n), jnp.float32),
            pltpu.VMEM((_NGRAPH, 1), jnp.float32),
        ],
    )(h, batch3, w_pad, b_pad)
    return out[:, :ntask]


_SC_NC = 2
_SC_NS = 16
_SC_G = 80
_NB = 25


def _sc_mesh():
    return plsc.VectorSubcoreMesh(core_axis_name="c", subcore_axis_name="s")


def _sc_segsum_edges(h, src, dst, zeros):
    n, emb = h.shape
    e = src.shape[0]
    e_per_w = e // (_SC_NC * _SC_NS)
    n_groups = e_per_w // _SC_G
    stripe = ((n // _SC_NS + 7) // 8) * 8
    last = n - stripe * (_SC_NS - 1)
    z2 = zeros[: stripe * emb].reshape(stripe, emb)
    nw = _SC_NC * _SC_NS
    src3 = src.reshape(nw, n_groups, _SC_G)
    dst3 = dst.reshape(nw, n_groups, _SC_G)

    def body(h_hbm, src_hbm, dst_hbm, z_hbm, out_hbm, acc, idxs, idxd, vals,
             sems, sems_s):
        c = lax.axis_index("c")
        s = lax.axis_index("s")
        wid = c * _SC_NS + s

        @pl.when(s < _SC_NS - 1)
        def _():
            pltpu.sync_copy(z_hbm, acc.at[pl.ds(s * stripe, stripe)])

        @pl.when(s == _SC_NS - 1)
        def _():
            pltpu.sync_copy(z_hbm.at[pl.ds(0, last)],
                            acc.at[pl.ds(s * stripe, last)])

        plsc.subcore_barrier()

        def block(b, carry):
            pltpu.sync_copy(src_hbm.at[wid, pl.ds(b * _NB, _NB)], idxs)
            pltpu.sync_copy(dst_hbm.at[wid, pl.ds(b * _NB, _NB)], idxd)
            _pipe_gather_scatter(h_hbm, idxs, idxd, _NB, vals, sems, sems_s, acc)
            return carry

        lax.fori_loop(0, n_groups // _NB, block, 0)
        plsc.subcore_barrier()

        @pl.when(s < _SC_NS - 1)
        def _():
            pltpu.sync_copy(acc.at[pl.ds(s * stripe, stripe)],
                            out_hbm.at[c, pl.ds(s * stripe, stripe)])

        @pl.when(s == _SC_NS - 1)
        def _():
            pltpu.sync_copy(acc.at[pl.ds(s * stripe, last)],
                            out_hbm.at[c, pl.ds(s * stripe, last)])

    f = pl.kernel(
        body,
        out_type=jax.ShapeDtypeStruct((_SC_NC, n, emb), jnp.float32),
        mesh=_sc_mesh(),
        compiler_params=pltpu.CompilerParams(use_tc_tiling_on_sc=False),
        scratch_types=[
            pltpu.VMEM_SHARED((n, emb), jnp.float32),
            pltpu.VMEM((_NB, _SC_G), jnp.int32),
            pltpu.VMEM((_NB, _SC_G), jnp.int32),
            pltpu.VMEM((2 * _SC_G, emb), jnp.float32),
            pltpu.SemaphoreType.DMA((2,)),
            pltpu.SemaphoreType.DMA((2,)),
        ],
    )
    return f(h, src3, dst3, z2)


def _add_split_body(a_ref, b_ref, o0, o1, o2, o3):
    y = a_ref[...] + b_ref[...]
    for q, o in enumerate((o0, o1, o2, o3)):
        o[...] = y[:, 32 * q:32 * (q + 1)]


def _add_split(a, b):
    m, n = a.shape
    grid = m // _BM
    return pl.pallas_call(
        _add_split_body,
        grid=(grid,),
        in_specs=[pl.BlockSpec((_BM, n), lambda i: (i, 0))] * 2,
        out_specs=[pl.BlockSpec((_BM, 32), lambda i: (i, 0))] * 4,
        out_shape=[jax.ShapeDtypeStruct((m, 32), jnp.float32)] * 4,
    )(a, b)


def _mm_relu_slabs_body(s0, s1, s2, s3, w0, w1, w2, w3, b_ref, o_ref):
    y = b_ref[...]
    for s_ref, w_ref in ((s0, w0), (s1, w1), (s2, w2), (s3, w3)):
        y = y + jnp.dot(s_ref[...], w_ref[...],
                        preferred_element_type=jnp.float32)
    o_ref[...] = jnp.maximum(y, 0.0)


def _mm_relu_slabs(slabs, w, b):
    m = slabs[0].shape[0]
    n = w.shape[1]
    w_slices = [w[32 * q:32 * (q + 1), :] for q in range(4)]
    grid = m // _BM
    return pl.pallas_call(
        _mm_relu_slabs_body,
        grid=(grid,),
        in_specs=[pl.BlockSpec((_BM, 32), lambda i: (i, 0))] * 4
        + [pl.BlockSpec((32, n), lambda i: (0, 0))] * 4
        + [pl.BlockSpec((n,), lambda i: (0,))],
        out_specs=pl.BlockSpec((_BM, n), lambda i: (i, 0)),
        out_shape=jax.ShapeDtypeStruct((m, n), jnp.float32),
    )(*slabs, *w_slices, b)


def _pipe_gather_scatter(table, idxs_v, idxd_v, n_groups, vals2, sems,
                         sems_s, acc):
    g = _SC_G

    def gather_start(k, slot):
        pltpu.async_copy(table.at[idxs_v.at[k]],
                         vals2.at[pl.ds(slot * g, g)], sems.at[slot])

    def gather_wait(k, slot):
        pltpu.make_async_copy(table.at[idxs_v.at[k]],
                              vals2.at[pl.ds(slot * g, g)],
                              sems.at[slot]).wait()

    def scatter_start(k, slot):
        pltpu.async_copy(vals2.at[pl.ds(slot * g, g)], acc.at[idxd_v.at[k]],
                         sems_s.at[slot], add=True)

    def scatter_wait(k, slot):
        pltpu.make_async_copy(vals2.at[pl.ds(slot * g, g)],
                              acc.at[idxd_v.at[k]], sems_s.at[slot]).wait()

    gather_start(0, 0)

    def step(k, carry):
        slot = lax.rem(k, 2)
        nslot = 1 - slot

        @pl.when(k >= 1)
        def _():
            scatter_wait(k - 1, nslot)

        @pl.when(k + 1 < n_groups)
        def _():
            gather_start(k + 1, nslot)

        gather_wait(k, slot)
        scatter_start(k, slot)
        return carry

    lax.fori_loop(0, n_groups, step, 0)
    scatter_wait(n_groups - 1, (n_groups - 1) % 2)


def _sc_gather_xs(h, lab, nodeidx, nodelabel):
    n, emb = h.shape
    nsub = nodeidx.shape[0]
    n_groups = nsub // _SC_G
    k_max = (n_groups + _SC_NC * _SC_NS - 1) // (_SC_NC * _SC_NS)

    def body(h_hbm, lab_hbm, ni_hbm, nl_hbm, xh_hbm, xl_hbm,
             idx1, idx2, vals, vals2):
        c = lax.axis_index("c")
        s = lax.axis_index("s")
        wid = c * _SC_NS + s

        def step(k, carry):
            g = wid * k_max + k

            @pl.when(g < n_groups)
            def _():
                base = g * _SC_G
                pltpu.sync_copy(ni_hbm.at[pl.ds(base, _SC_G)], idx1)
                pltpu.sync_copy(nl_hbm.at[pl.ds(base, _SC_G)], idx2)
                pltpu.sync_copy(h_hbm.at[idx1], vals)
                pltpu.sync_copy(vals, xh_hbm.at[pl.ds(base, _SC_G)])
                pltpu.sync_copy(lab_hbm.at[idx2], vals2)
                pltpu.sync_copy(vals2, xl_hbm.at[pl.ds(base, _SC_G)])

            return carry

        lax.fori_loop(0, k_max, step, 0)

    f = pl.kernel(
        body,
        out_type=(jax.ShapeDtypeStruct((nsub, emb), jnp.float32),
                  jax.ShapeDtypeStruct((nsub, emb), jnp.float32)),
        mesh=_sc_mesh(),
        scratch_types=[
            pltpu.VMEM((_SC_G,), jnp.int32),
            pltpu.VMEM((_SC_G,), jnp.int32),
            pltpu.VMEM((_SC_G, emb), jnp.float32),
            pltpu.VMEM((_SC_G, emb), jnp.float32),
        ],
    )
    return f(h, lab, nodeidx, nodelabel)


def _sc_segsum_sub(xs_slabs, src, dst, zeros):
    nsub = xs_slabs[0].shape[0]
    e = src.shape[0]
    e_per_s = e // _SC_NS
    n_groups = e_per_s // _SC_G
    stripe = ((nsub // _SC_NS + 7) // 8) * 8
    last = nsub - stripe * (_SC_NS - 1)
    z2 = zeros[: stripe * 32].reshape(stripe, 32)
    src3 = src.reshape(_SC_NS, n_groups, _SC_G)
    dst3 = dst.reshape(_SC_NS, n_groups, _SC_G)

    def body(x0, x1, x2, x3, src_hbm, dst_hbm, z_hbm, a0, a1, a2, a3,
             acc, idxs, idxd, vals, sems, sems_s):
        c = lax.axis_index("c")
        s = lax.axis_index("s")
        xs_t = (x0, x1, x2, x3)
        ag_t = (a0, a1, a2, a3)
        for qi in range(2):
            for cc in range(_SC_NC):
                @pl.when(c == cc)
                def _(qi=qi, cc=cc):
                    xq = xs_t[2 * cc + qi]
                    aq = ag_t[2 * cc + qi]

                    @pl.when(s < _SC_NS - 1)
                    def _():
                        pltpu.sync_copy(z_hbm,
                                        acc.at[pl.ds(s * stripe, stripe)])

                    @pl.when(s == _SC_NS - 1)
                    def _():
                        pltpu.sync_copy(z_hbm.at[pl.ds(0, last)],
                                        acc.at[pl.ds(s * stripe, last)])

                    plsc.subcore_barrier()

                    def block(b, carry, xq=xq):
                        pltpu.sync_copy(src_hbm.at[s, pl.ds(b * _NB, _NB)],
                                        idxs)
                        pltpu.sync_copy(dst_hbm.at[s, pl.ds(b * _NB, _NB)],
                                        idxd)
                        _pipe_gather_scatter(xq, idxs, idxd, _NB,
                                             vals, sems, sems_s, acc)
                        return carry

                    lax.fori_loop(0, n_groups // _NB, block, 0)
                    plsc.subcore_barrier()

                    @pl.when(s < _SC_NS - 1)
                    def _():
                        pltpu.sync_copy(acc.at[pl.ds(s * stripe, stripe)],
                                        aq.at[pl.ds(s * stripe, stripe)])

                    @pl.when(s == _SC_NS - 1)
                    def _():
                        pltpu.sync_copy(acc.at[pl.ds(s * stripe, last)],
                                        aq.at[pl.ds(s * stripe, last)])

                    plsc.subcore_barrier()

    f = pl.kernel(
        body,
        out_type=tuple(jax.ShapeDtypeStruct((nsub, 32), jnp.float32)
                       for _ in range(4)),
        mesh=_sc_mesh(),
        compiler_params=pltpu.CompilerParams(use_tc_tiling_on_sc=False),
        scratch_types=[
            pltpu.VMEM_SHARED((nsub, 32), jnp.float32),
            pltpu.VMEM((_NB, _SC_G), jnp.int32),
            pltpu.VMEM((_NB, _SC_G), jnp.int32),
            pltpu.VMEM((2 * _SC_G, 32), jnp.float32),
            pltpu.SemaphoreType.DMA((2,)),
            pltpu.SemaphoreType.DMA((2,)),
        ],
    )
    return f(*xs_slabs, src3, dst3, z2)


def _sc_scatter_mean(h_sub, nodeidx, zeros, n_out):
    nsub, emb = h_sub.shape
    n_groups = nsub // _SC_G
    k_max = (n_groups + _SC_NC * _SC_NS - 1) // (_SC_NC * _SC_NS)
    stripe = ((n_out // _SC_NS + 7) // 8) * 8
    last = n_out - stripe * (_SC_NS - 1)
    z_s = zeros[: stripe * emb].reshape(stripe, emb)
    ones = jnp.ones((_SC_G, emb), jnp.float32)

    def body(hs_hbm, ni_hbm, z_hbm, ones_hbm, sum_hbm, cnt_hbm,
             acc, idxd, vals, ones_v):
        c = lax.axis_index("c")
        s = lax.axis_index("s")
        wid = c * _SC_NS + s
        pltpu.sync_copy(ones_hbm, ones_v)

        def zero_acc():
            @pl.when(s < _SC_NS - 1)
            def _():
                pltpu.sync_copy(z_hbm, acc.at[pl.ds(s * stripe, stripe)])

            @pl.when(s == _SC_NS - 1)
            def _():
                pltpu.sync_copy(z_hbm.at[pl.ds(0, last)],
                                acc.at[pl.ds(s * stripe, last)])

        def flush_acc(dst_hbm):
            @pl.when(s < _SC_NS - 1)
            def _():
                pltpu.sync_copy(acc.at[pl.ds(s * stripe, stripe)],
                                dst_hbm.at[c, pl.ds(s * stripe, stripe)])

            @pl.when(s == _SC_NS - 1)
            def _():
                pltpu.sync_copy(acc.at[pl.ds(s * stripe, last)],
                                dst_hbm.at[c, pl.ds(s * stripe, last)])

        def scan_groups(do_group):
            def step(k, carry):
                g = wid * k_max + k

                @pl.when(g < n_groups)
                def _():
                    do_group(g * _SC_G)

                return carry

            lax.fori_loop(0, k_max, step, 0)

        zero_acc()
        plsc.subcore_barrier()

        def sum_group(base):
            pltpu.sync_copy(ni_hbm.at[pl.ds(base, _SC_G)], idxd)
            pltpu.sync_copy(hs_hbm.at[pl.ds(base, _SC_G)], vals)
            pltpu.sync_copy(vals, acc.at[idxd], add=True)

        scan_groups(sum_group)
        plsc.subcore_barrier()
        flush_acc(sum_hbm)
        plsc.subcore_barrier()

        zero_acc()
        plsc.subcore_barrier()

        def cnt_group(base):
            pltpu.sync_copy(ni_hbm.at[pl.ds(base, _SC_G)], idxd)
            pltpu.sync_copy(ones_v, acc.at[idxd], add=True)

        scan_groups(cnt_group)
        plsc.subcore_barrier()
        flush_acc(cnt_hbm)

    f = pl.kernel(
        body,
        out_type=(jax.ShapeDtypeStruct((_SC_NC, n_out, emb), jnp.float32),
                  jax.ShapeDtypeStruct((_SC_NC, n_out, emb), jnp.float32)),
        mesh=_sc_mesh(),
        scratch_types=[
            pltpu.VMEM_SHARED((n_out, emb), jnp.float32),
            pltpu.VMEM((_SC_G,), jnp.int32),
            pltpu.VMEM((_SC_G, emb), jnp.float32),
            pltpu.VMEM((_SC_G, emb), jnp.float32),
        ],
    )
    return f(h_sub, nodeidx, z_s, ones)


def kernel(x, edge_index, subg_nodeidx, subg_nodelabel, subg_edge_index, batch,
           W_enc, b_enc, label_emb, W_l, b_l, W_g, b_g, W_out, b_out):
    zeros = jnp.zeros((81920,), jnp.float32)

    h = _mm_relu([x], W_enc, b_enc)

    xh, xl = _sc_gather_xs(h, label_emb, subg_nodeidx, subg_nodelabel)
    xs_slabs = _add_split(xh, xl)

    agg_slabs = _sc_segsum_sub(xs_slabs, subg_edge_index[0],
                               subg_edge_index[1], zeros)

    h_sub = _mm_relu_slabs(agg_slabs, W_l, b_l)

    sums, cnts = _sc_scatter_mean(h_sub, subg_nodeidx, zeros, _N)
    h1 = _mean_residual(h, sums[0], sums[1], cnts[0], cnts[1])

    gp = _sc_segsum_edges(h1, edge_index[0], edge_index[1], zeros)

    h2 = _mm_relu([gp[0], gp[1]], W_g, b_g, residual=h1)

    return _pool_predict(h2, batch, W_out, b_out)


# speedup vs baseline: 4.9071x; 1.0016x over previous
import functools

import jax
import jax.numpy as jnp
from jax import lax
from jax.experimental import pallas as pl
from jax.experimental.pallas import tpu as pltpu
from jax.experimental.pallas import tpu_sc as plsc

_N = 10000
_E = 320000
_NSUB = 40000
_ESUB = 320000
_NGRAPH = 64
_EMB = 128

_BM = 2000


def _mm_relu_body(n_parts, residual, *refs):
    *x_refs, w_ref, b_ref, o_ref = refs
    acc = x_refs[0][...]
    for r in x_refs[1:n_parts]:
        acc = acc + r[...]
    y = jnp.dot(acc, w_ref[...], preferred_element_type=jnp.float32)
    y = jnp.maximum(y + b_ref[...], 0.0)
    if residual:
        y = x_refs[n_parts][...] + y
    o_ref[...] = y


def _mm_relu(parts, w, b, residual=None):
    m = parts[0].shape[0]
    k = parts[0].shape[1]
    n = w.shape[1]
    inputs = list(parts) + ([residual] if residual is not None else [])
    grid = m // _BM
    body = functools.partial(_mm_relu_body, len(parts), residual is not None)
    return pl.pallas_call(
        body,
        grid=(grid,),
        in_specs=[pl.BlockSpec((_BM, k), lambda i: (i, 0)) for _ in inputs]
        + [
            pl.BlockSpec((k, n), lambda i: (0, 0)),
            pl.BlockSpec((n,), lambda i: (0,)),
        ],
        out_specs=pl.BlockSpec((_BM, n), lambda i: (i, 0)),
        out_shape=jax.ShapeDtypeStruct((m, n), jnp.float32),
    )(*inputs, w, b)


def _mean_residual_body(h_ref, s0_ref, s1_ref, c0_ref, c1_ref, o_ref):
    cnt = jnp.maximum((c0_ref[...] + c1_ref[...])[:, 0:1], 1.0)
    o_ref[...] = h_ref[...] + (s0_ref[...] + s1_ref[...]) / cnt


def _mean_residual(h, s0, s1, c0, c1):
    m, n = h.shape
    grid = m // _BM
    return pl.pallas_call(
        _mean_residual_body,
        grid=(grid,),
        in_specs=[pl.BlockSpec((_BM, n), lambda i: (i, 0)) for _ in range(5)],
        out_specs=pl.BlockSpec((_BM, n), lambda i: (i, 0)),
        out_shape=jax.ShapeDtypeStruct((m, n), jnp.float32),
    )(h, s0, s1, c0, c1)


def _pool_body(h_ref, batch_ref, w_ref, b_ref, o_ref, acc_ref, cnt_ref):
    i = pl.program_id(0)

    @pl.when(i == 0)
    def _():
        acc_ref[...] = jnp.zeros_like(acc_ref)
        cnt_ref[...] = jnp.zeros_like(cnt_ref)

    seg = batch_ref[...].reshape(1, _BM)
    gids = lax.broadcasted_iota(jnp.int32, (_NGRAPH, _BM), 0)
    mask = (gids == seg).astype(jnp.float32)
    acc_ref[...] += jnp.dot(mask, h_ref[...], preferred_element_type=jnp.float32)
    cnt_ref[...] += jnp.sum(mask, axis=1, keepdims=True)

    @pl.when(i == pl.num_programs(0) - 1)
    def _():
        hg = acc_ref[...] / jnp.maximum(cnt_ref[...], 1.0)
        o_ref[...] = jnp.dot(hg, w_ref[...], preferred_element_type=jnp.float32) + b_ref[...]


def _pool_predict(h, batch, w_out, b_out):
    m, n = h.shape
    ntask = w_out.shape[1]
    w_pad = jnp.zeros((n, 128), jnp.float32).at[:, :ntask].set(w_out)
    b_pad = jnp.zeros((128,), jnp.float32).at[:ntask].set(b_out)
    batch3 = batch.reshape(m // _BM, 1, _BM)
    grid = m // _BM
    out = pl.pallas_call(
        _pool_body,
        grid=(grid,),
        in_specs=[
            pl.BlockSpec((_BM, n), lambda i: (i, 0)),
            pl.BlockSpec((1, 1, _BM), lambda i: (i, 0, 0)),
            pl.BlockSpec((n, 128), lambda i: (0, 0)),
            pl.BlockSpec((128,), lambda i: (0,)),
        ],
        out_specs=pl.BlockSpec((_NGRAPH, 128), lambda i: (0, 0)),
        out_shape=jax.ShapeDtypeStruct((_NGRAPH, 128), jnp.float32),
        scratch_shapes=[
            pltpu.VMEM((_NGRAPH, n), jnp.float32),
            pltpu.VMEM((_NGRAPH, 1), jnp.float32),
        ],
    )(h, batch3, w_pad, b_pad)
    return out[:, :ntask]


_SC_NC = 2
_SC_NS = 16
_SC_G = 80
_NB = 25


def _sc_mesh():
    return plsc.VectorSubcoreMesh(core_axis_name="c", subcore_axis_name="s")


def _sc_segsum_edges(h, src, dst, zeros):
    n, emb = h.shape
    e = src.shape[0]
    e_per_w = e // (_SC_NC * _SC_NS)
    n_groups = e_per_w // _SC_G
    stripe = ((n // _SC_NS + 7) // 8) * 8
    last = n - stripe * (_SC_NS - 1)
    z2 = zeros[: stripe * emb].reshape(stripe, emb)
    nw = _SC_NC * _SC_NS
    src3 = src.reshape(nw, n_groups, _SC_G)
    dst3 = dst.reshape(nw, n_groups, _SC_G)

    def body(h_hbm, src_hbm, dst_hbm, z_hbm, out_hbm, acc, idxs, idxd, vals,
             sems, sems_s):
        c = lax.axis_index("c")
        s = lax.axis_index("s")
        wid = c * _SC_NS + s

        @pl.when(s < _SC_NS - 1)
        def _():
            pltpu.sync_copy(z_hbm, acc.at[pl.ds(s * stripe, stripe)])

        @pl.when(s == _SC_NS - 1)
        def _():
            pltpu.sync_copy(z_hbm.at[pl.ds(0, last)],
                            acc.at[pl.ds(s * stripe, last)])

        plsc.subcore_barrier()

        def block(b, carry):
            pltpu.sync_copy(src_hbm.at[wid, pl.ds(b * _NB, _NB)], idxs)
            pltpu.sync_copy(dst_hbm.at[wid, pl.ds(b * _NB, _NB)], idxd)
            _pipe_gather_scatter(h_hbm, idxs, idxd, _NB, vals, sems, sems_s, acc)
            return carry

        lax.fori_loop(0, n_groups // _NB, block, 0)
        plsc.subcore_barrier()

        @pl.when(s < _SC_NS - 1)
        def _():
            pltpu.sync_copy(acc.at[pl.ds(s * stripe, stripe)],
                            out_hbm.at[c, pl.ds(s * stripe, stripe)])

        @pl.when(s == _SC_NS - 1)
        def _():
            pltpu.sync_copy(acc.at[pl.ds(s * stripe, last)],
                            out_hbm.at[c, pl.ds(s * stripe, last)])

    f = pl.kernel(
        body,
        out_type=jax.ShapeDtypeStruct((_SC_NC, n, emb), jnp.float32),
        mesh=_sc_mesh(),
        compiler_params=pltpu.CompilerParams(use_tc_tiling_on_sc=False),
        scratch_types=[
            pltpu.VMEM_SHARED((n, emb), jnp.float32),
            pltpu.VMEM((_NB, _SC_G), jnp.int32),
            pltpu.VMEM((_NB, _SC_G), jnp.int32),
            pltpu.VMEM((2 * _SC_G, emb), jnp.float32),
            pltpu.SemaphoreType.DMA((2,)),
            pltpu.SemaphoreType.DMA((2,)),
        ],
    )
    return f(h, src3, dst3, z2)


def _add_split_body(a_ref, b_ref, o0, o1, o2, o3):
    y = a_ref[...] + b_ref[...]
    for q, o in enumerate((o0, o1, o2, o3)):
        o[...] = y[:, 32 * q:32 * (q + 1)]


def _add_split(a, b):
    m, n = a.shape
    grid = m // _BM
    return pl.pallas_call(
        _add_split_body,
        grid=(grid,),
        in_specs=[pl.BlockSpec((_BM, n), lambda i: (i, 0))] * 2,
        out_specs=[pl.BlockSpec((_BM, 32), lambda i: (i, 0))] * 4,
        out_shape=[jax.ShapeDtypeStruct((m, 32), jnp.float32)] * 4,
    )(a, b)


def _mm_relu_slabs_body(s0, s1, s2, s3, w0, w1, w2, w3, b_ref, o_ref):
    y = b_ref[...]
    for s_ref, w_ref in ((s0, w0), (s1, w1), (s2, w2), (s3, w3)):
        y = y + jnp.dot(s_ref[...], w_ref[...],
                        preferred_element_type=jnp.float32)
    o_ref[...] = jnp.maximum(y, 0.0)


def _mm_relu_slabs(slabs, w, b):
    m = slabs[0].shape[0]
    n = w.shape[1]
    w_slices = [w[32 * q:32 * (q + 1), :] for q in range(4)]
    grid = m // _BM
    return pl.pallas_call(
        _mm_relu_slabs_body,
        grid=(grid,),
        in_specs=[pl.BlockSpec((_BM, 32), lambda i: (i, 0))] * 4
        + [pl.BlockSpec((32, n), lambda i: (0, 0))] * 4
        + [pl.BlockSpec((n,), lambda i: (0,))],
        out_specs=pl.BlockSpec((_BM, n), lambda i: (i, 0)),
        out_shape=jax.ShapeDtypeStruct((m, n), jnp.float32),
    )(*slabs, *w_slices, b)


def _pipe_gather_scatter(table, idxs_v, idxd_v, n_groups, vals2, sems,
                         sems_s, acc):
    g = _SC_G

    def gather_start(k, slot):
        pltpu.async_copy(table.at[idxs_v.at[k]],
                         vals2.at[pl.ds(slot * g, g)], sems.at[slot])

    def gather_wait(k, slot):
        pltpu.make_async_copy(table.at[idxs_v.at[k]],
                              vals2.at[pl.ds(slot * g, g)],
                              sems.at[slot]).wait()

    def scatter_start(k, slot):
        pltpu.async_copy(vals2.at[pl.ds(slot * g, g)], acc.at[idxd_v.at[k]],
                         sems_s.at[slot], add=True)

    def scatter_wait(k, slot):
        pltpu.make_async_copy(vals2.at[pl.ds(slot * g, g)],
                              acc.at[idxd_v.at[k]], sems_s.at[slot]).wait()

    gather_start(0, 0)

    def step(k, carry):
        slot = lax.rem(k, 2)
        nslot = 1 - slot

        @pl.when(k >= 1)
        def _():
            scatter_wait(k - 1, nslot)

        @pl.when(k + 1 < n_groups)
        def _():
            gather_start(k + 1, nslot)

        gather_wait(k, slot)
        scatter_start(k, slot)
        return carry

    lax.fori_loop(0, n_groups, step, 0)
    scatter_wait(n_groups - 1, (n_groups - 1) % 2)


def _sc_gather_xs(h, lab, nodeidx, nodelabel):
    n, emb = h.shape
    nsub = nodeidx.shape[0]
    n_groups = nsub // _SC_G
    nw = _SC_NC * _SC_NS
    k_max = (n_groups + nw - 1) // nw
    npad = nw * k_max
    ni2 = jnp.zeros((npad, _SC_G), jnp.int32)
    ni2 = ni2.at[:n_groups].set(nodeidx.reshape(n_groups, _SC_G))
    nl2 = jnp.zeros((npad, _SC_G), jnp.int32)
    nl2 = nl2.at[:n_groups].set(nodelabel.reshape(n_groups, _SC_G))
    g_ = _SC_G

    def body(h_hbm, lab_hbm, ni_hbm, nl_hbm, xh_hbm, xl_hbm,
             idx1, idx2, vals, vals2, semg, semg2, semw, semw2):
        c = lax.axis_index("c")
        s = lax.axis_index("s")
        wid = c * _SC_NS + s
        pltpu.sync_copy(ni_hbm.at[pl.ds(wid * k_max, k_max)], idx1)
        pltpu.sync_copy(nl_hbm.at[pl.ds(wid * k_max, k_max)], idx2)

        def gather_start(k, slot):
            pltpu.async_copy(h_hbm.at[idx1.at[k]],
                             vals.at[pl.ds(slot * g_, g_)], semg.at[slot])
            pltpu.async_copy(lab_hbm.at[idx2.at[k]],
                             vals2.at[pl.ds(slot * g_, g_)], semg2.at[slot])

        def gather_wait(k, slot):
            pltpu.make_async_copy(h_hbm.at[idx1.at[k]],
                                  vals.at[pl.ds(slot * g_, g_)],
                                  semg.at[slot]).wait()
            pltpu.make_async_copy(lab_hbm.at[idx2.at[k]],
                                  vals2.at[pl.ds(slot * g_, g_)],
                                  semg2.at[slot]).wait()

        def write_start(k, slot):
            base = (wid * k_max + k) * g_
            pltpu.async_copy(vals.at[pl.ds(slot * g_, g_)],
                             xh_hbm.at[pl.ds(base, g_)], semw.at[slot])
            pltpu.async_copy(vals2.at[pl.ds(slot * g_, g_)],
                             xl_hbm.at[pl.ds(base, g_)], semw2.at[slot])

        def write_wait(k, slot):
            base = (wid * k_max + k) * g_
            pltpu.make_async_copy(vals.at[pl.ds(slot * g_, g_)],
                                  xh_hbm.at[pl.ds(base, g_)],
                                  semw.at[slot]).wait()
            pltpu.make_async_copy(vals2.at[pl.ds(slot * g_, g_)],
                                  xl_hbm.at[pl.ds(base, g_)],
                                  semw2.at[slot]).wait()

        gather_start(0, 0)

        def step(k, carry):
            slot = lax.rem(k, 2)
            nslot = 1 - slot
            g = wid * k_max + k

            @pl.when((k >= 1) & (g - 1 < n_groups))
            def _():
                write_wait(k - 1, nslot)

            @pl.when((k + 1 < k_max) & (g + 1 < n_groups))
            def _():
                gather_start(k + 1, nslot)

            @pl.when(g < n_groups)
            def _():
                gather_wait(k, slot)
                write_start(k, slot)

            return carry

        lax.fori_loop(0, k_max, step, 0)
        last_g = wid * k_max + k_max - 1

        @pl.when(last_g < n_groups)
        def _():
            write_wait(k_max - 1, (k_max - 1) % 2)

    f = pl.kernel(
        body,
        out_type=(jax.ShapeDtypeStruct((nsub, emb), jnp.float32),
                  jax.ShapeDtypeStruct((nsub, emb), jnp.float32)),
        mesh=_sc_mesh(),
        scratch_types=[
            pltpu.VMEM((k_max, _SC_G), jnp.int32),
            pltpu.VMEM((k_max, _SC_G), jnp.int32),
            pltpu.VMEM((2 * _SC_G, emb), jnp.float32),
            pltpu.VMEM((2 * _SC_G, emb), jnp.float32),
            pltpu.SemaphoreType.DMA((2,)),
            pltpu.SemaphoreType.DMA((2,)),
            pltpu.SemaphoreType.DMA((2,)),
            pltpu.SemaphoreType.DMA((2,)),
        ],
    )
    return f(h, lab, ni2, nl2)


def _sc_segsum_sub(xs_slabs, src, dst, zeros):
    nsub = xs_slabs[0].shape[0]
    e = src.shape[0]
    e_per_s = e // _SC_NS
    n_groups = e_per_s // _SC_G
    stripe = ((nsub // _SC_NS + 7) // 8) * 8
    last = nsub - stripe * (_SC_NS - 1)
    z2 = zeros[: stripe * 32].reshape(stripe, 32)
    src3 = src.reshape(_SC_NS, n_groups, _SC_G)
    dst3 = dst.reshape(_SC_NS, n_groups, _SC_G)

    def body(x0, x1, x2, x3, src_hbm, dst_hbm, z_hbm, a0, a1, a2, a3,
             acc, idxs, idxd, vals, sems, sems_s):
        c = lax.axis_index("c")
        s = lax.axis_index("s")
        xs_t = (x0, x1, x2, x3)
        ag_t = (a0, a1, a2, a3)
        for qi in range(2):
            for cc in range(_SC_NC):
                @pl.when(c == cc)
                def _(qi=qi, cc=cc):
                    xq = xs_t[2 * cc + qi]
                    aq = ag_t[2 * cc + qi]

                    @pl.when(s < _SC_NS - 1)
                    def _():
                        pltpu.sync_copy(z_hbm,
                                        acc.at[pl.ds(s * stripe, stripe)])

                    @pl.when(s == _SC_NS - 1)
                    def _():
                        pltpu.sync_copy(z_hbm.at[pl.ds(0, last)],
                                        acc.at[pl.ds(s * stripe, last)])

                    plsc.subcore_barrier()

                    def block(b, carry, xq=xq):
                        pltpu.sync_copy(src_hbm.at[s, pl.ds(b * _NB, _NB)],
                                        idxs)
                        pltpu.sync_copy(dst_hbm.at[s, pl.ds(b * _NB, _NB)],
                                        idxd)
                        _pipe_gather_scatter(xq, idxs, idxd, _NB,
                                             vals, sems, sems_s, acc)
                        return carry

                    lax.fori_loop(0, n_groups // _NB, block, 0)
                    plsc.subcore_barrier()

                    @pl.when(s < _SC_NS - 1)
                    def _():
                        pltpu.sync_copy(acc.at[pl.ds(s * stripe, stripe)],
                                        aq.at[pl.ds(s * stripe, stripe)])

                    @pl.when(s == _SC_NS - 1)
                    def _():
                        pltpu.sync_copy(acc.at[pl.ds(s * stripe, last)],
                                        aq.at[pl.ds(s * stripe, last)])

                    plsc.subcore_barrier()

    f = pl.kernel(
        body,
        out_type=tuple(jax.ShapeDtypeStruct((nsub, 32), jnp.float32)
                       for _ in range(4)),
        mesh=_sc_mesh(),
        compiler_params=pltpu.CompilerParams(use_tc_tiling_on_sc=False),
        scratch_types=[
            pltpu.VMEM_SHARED((nsub, 32), jnp.float32),
            pltpu.VMEM((_NB, _SC_G), jnp.int32),
            pltpu.VMEM((_NB, _SC_G), jnp.int32),
            pltpu.VMEM((2 * _SC_G, 32), jnp.float32),
            pltpu.SemaphoreType.DMA((2,)),
            pltpu.SemaphoreType.DMA((2,)),
        ],
    )
    return f(*xs_slabs, src3, dst3, z2)


def _sc_scatter_mean(h_sub, nodeidx, zeros, n_out):
    nsub, emb = h_sub.shape
    n_groups = nsub // _SC_G
    k_max = (n_groups + _SC_NC * _SC_NS - 1) // (_SC_NC * _SC_NS)
    stripe = ((n_out // _SC_NS + 7) // 8) * 8
    last = n_out - stripe * (_SC_NS - 1)
    z_s = zeros[: stripe * emb].reshape(stripe, emb)
    ones = jnp.ones((_SC_G, emb), jnp.float32)

    def body(hs_hbm, ni_hbm, z_hbm, ones_hbm, sum_hbm, cnt_hbm,
             acc, idxd, vals, ones_v):
        c = lax.axis_index("c")
        s = lax.axis_index("s")
        wid = c * _SC_NS + s
        pltpu.sync_copy(ones_hbm, ones_v)

        def zero_acc():
            @pl.when(s < _SC_NS - 1)
            def _():
                pltpu.sync_copy(z_hbm, acc.at[pl.ds(s * stripe, stripe)])

            @pl.when(s == _SC_NS - 1)
            def _():
                pltpu.sync_copy(z_hbm.at[pl.ds(0, last)],
                                acc.at[pl.ds(s * stripe, last)])

        def flush_acc(dst_hbm):
            @pl.when(s < _SC_NS - 1)
            def _():
                pltpu.sync_copy(acc.at[pl.ds(s * stripe, stripe)],
                                dst_hbm.at[c, pl.ds(s * stripe, stripe)])

            @pl.when(s == _SC_NS - 1)
            def _():
                pltpu.sync_copy(acc.at[pl.ds(s * stripe, last)],
                                dst_hbm.at[c, pl.ds(s * stripe, last)])

        def scan_groups(do_group):
            def step(k, carry):
                g = wid * k_max + k

                @pl.when(g < n_groups)
                def _():
                    do_group(g * _SC_G)

                return carry

            lax.fori_loop(0, k_max, step, 0)

        zero_acc()
        plsc.subcore_barrier()

        def sum_group(base):
            pltpu.sync_copy(ni_hbm.at[pl.ds(base, _SC_G)], idxd)
            pltpu.sync_copy(hs_hbm.at[pl.ds(base, _SC_G)], vals)
            pltpu.sync_copy(vals, acc.at[idxd], add=True)

        scan_groups(sum_group)
        plsc.subcore_barrier()
        flush_acc(sum_hbm)
        plsc.subcore_barrier()

        zero_acc()
        plsc.subcore_barrier()

        def cnt_group(base):
            pltpu.sync_copy(ni_hbm.at[pl.ds(base, _SC_G)], idxd)
            pltpu.sync_copy(ones_v, acc.at[idxd], add=True)

        scan_groups(cnt_group)
        plsc.subcore_barrier()
        flush_acc(cnt_hbm)

    f = pl.kernel(
        body,
        out_type=(jax.ShapeDtypeStruct((_SC_NC, n_out, emb), jnp.float32),
                  jax.ShapeDtypeStruct((_SC_NC, n_out, emb), jnp.float32)),
        mesh=_sc_mesh(),
        scratch_types=[
            pltpu.VMEM_SHARED((n_out, emb), jnp.float32),
            pltpu.VMEM((_SC_G,), jnp.int32),
            pltpu.VMEM((_SC_G, emb), jnp.float32),
            pltpu.VMEM((_SC_G, emb), jnp.float32),
        ],
    )
    return f(h_sub, nodeidx, z_s, ones)


def kernel(x, edge_index, subg_nodeidx, subg_nodelabel, subg_edge_index, batch,
           W_enc, b_enc, label_emb, W_l, b_l, W_g, b_g, W_out, b_out):
    zeros = jnp.zeros((81920,), jnp.float32)

    h = _mm_relu([x], W_enc, b_enc)

    xh, xl = _sc_gather_xs(h, label_emb, subg_nodeidx, subg_nodelabel)
    xs_slabs = _add_split(xh, xl)

    agg_slabs = _sc_segsum_sub(xs_slabs, subg_edge_index[0],
                               subg_edge_index[1], zeros)

    h_sub = _mm_relu_slabs(agg_slabs, W_l, b_l)

    sums, cnts = _sc_scatter_mean(h_sub, subg_nodeidx, zeros, _N)
    h1 = _mean_residual(h, sums[0], sums[1], cnts[0], cnts[1])

    gp = _sc_segsum_edges(h1, edge_index[0], edge_index[1], zeros)

    h2 = _mm_relu([gp[0], gp[1]], W_g, b_g, residual=h1)

    return _pool_predict(h2, batch, W_out, b_out)


# speedup vs baseline: 5.4049x; 1.1014x over previous
import functools

import jax
import jax.numpy as jnp
from jax import lax
from jax.experimental import pallas as pl
from jax.experimental.pallas import tpu as pltpu
from jax.experimental.pallas import tpu_sc as plsc

_N = 10000
_E = 320000
_NSUB = 40000
_ESUB = 320000
_NGRAPH = 64
_EMB = 128

_BM = 2000


def _mm_relu_body(n_parts, residual, *refs):
    *x_refs, w_ref, b_ref, o_ref = refs
    acc = x_refs[0][...]
    for r in x_refs[1:n_parts]:
        acc = acc + r[...]
    y = jnp.dot(acc, w_ref[...], preferred_element_type=jnp.float32)
    y = jnp.maximum(y + b_ref[...], 0.0)
    if residual:
        y = x_refs[n_parts][...] + y
    o_ref[...] = y


def _mm_relu(parts, w, b, residual=None):
    m = parts[0].shape[0]
    k = parts[0].shape[1]
    n = w.shape[1]
    inputs = list(parts) + ([residual] if residual is not None else [])
    grid = m // _BM
    body = functools.partial(_mm_relu_body, len(parts), residual is not None)
    return pl.pallas_call(
        body,
        grid=(grid,),
        in_specs=[pl.BlockSpec((_BM, k), lambda i: (i, 0)) for _ in inputs]
        + [
            pl.BlockSpec((k, n), lambda i: (0, 0)),
            pl.BlockSpec((n,), lambda i: (0,)),
        ],
        out_specs=pl.BlockSpec((_BM, n), lambda i: (i, 0)),
        out_shape=jax.ShapeDtypeStruct((m, n), jnp.float32),
    )(*inputs, w, b)


def _mean_residual_body(h_ref, s0_ref, s1_ref, c0_ref, c1_ref, o_ref):
    cnt = jnp.maximum((c0_ref[...] + c1_ref[...])[:, 0:1], 1.0)
    o_ref[...] = h_ref[...] + (s0_ref[...] + s1_ref[...]) / cnt


def _mean_residual(h, s0, s1, c0, c1):
    m, n = h.shape
    grid = m // _BM
    return pl.pallas_call(
        _mean_residual_body,
        grid=(grid,),
        in_specs=[pl.BlockSpec((_BM, n), lambda i: (i, 0)) for _ in range(5)],
        out_specs=pl.BlockSpec((_BM, n), lambda i: (i, 0)),
        out_shape=jax.ShapeDtypeStruct((m, n), jnp.float32),
    )(h, s0, s1, c0, c1)


def _pool_body(h_ref, batch_ref, w_ref, b_ref, o_ref, acc_ref, cnt_ref):
    i = pl.program_id(0)

    @pl.when(i == 0)
    def _():
        acc_ref[...] = jnp.zeros_like(acc_ref)
        cnt_ref[...] = jnp.zeros_like(cnt_ref)

    seg = batch_ref[...].reshape(1, _BM)
    gids = lax.broadcasted_iota(jnp.int32, (_NGRAPH, _BM), 0)
    mask = (gids == seg).astype(jnp.float32)
    acc_ref[...] += jnp.dot(mask, h_ref[...], preferred_element_type=jnp.float32)
    cnt_ref[...] += jnp.sum(mask, axis=1, keepdims=True)

    @pl.when(i == pl.num_programs(0) - 1)
    def _():
        hg = acc_ref[...] / jnp.maximum(cnt_ref[...], 1.0)
        o_ref[...] = jnp.dot(hg, w_ref[...], preferred_element_type=jnp.float32) + b_ref[...]


def _pool_predict(h, batch, w_out, b_out):
    m, n = h.shape
    ntask = w_out.shape[1]
    w_pad = jnp.zeros((n, 128), jnp.float32).at[:, :ntask].set(w_out)
    b_pad = jnp.zeros((128,), jnp.float32).at[:ntask].set(b_out)
    batch3 = batch.reshape(m // _BM, 1, _BM)
    grid = m // _BM
    out = pl.pallas_call(
        _pool_body,
        grid=(grid,),
        in_specs=[
            pl.BlockSpec((_BM, n), lambda i: (i, 0)),
            pl.BlockSpec((1, 1, _BM), lambda i: (i, 0, 0)),
            pl.BlockSpec((n, 128), lambda i: (0, 0)),
            pl.BlockSpec((128,), lambda i: (0,)),
        ],
        out_specs=pl.BlockSpec((_NGRAPH, 128), lambda i: (0, 0)),
        out_shape=jax.ShapeDtypeStruct((_NGRAPH, 128), jnp.float32),
        scratch_shapes=[
            pltpu.VMEM((_NGRAPH, n), jnp.float32),
            pltpu.VMEM((_NGRAPH, 1), jnp.float32),
        ],
    )(h, batch3, w_pad, b_pad)
    return out[:, :ntask]


_SC_NC = 2
_SC_NS = 16
_SC_G = 80
_NB = 25
_GE = 125
_NBE = 20


def _sc_mesh():
    return plsc.VectorSubcoreMesh(core_axis_name="c", subcore_axis_name="s")


def _sc_segsum_edges(h, src, dst, zeros):
    n, emb = h.shape
    e = src.shape[0]
    e_per_w = e // (_SC_NC * _SC_NS)
    n_groups = e_per_w // _GE
    stripe = ((n // _SC_NS + 7) // 8) * 8
    last = n - stripe * (_SC_NS - 1)
    z2 = zeros[: stripe * emb].reshape(stripe, emb)
    nw = _SC_NC * _SC_NS
    src3 = src.reshape(nw, n_groups, _GE)
    dst3 = dst.reshape(nw, n_groups, _GE)

    def body(h_hbm, src_hbm, dst_hbm, z_hbm, out_hbm, acc, idxs, idxd, vals,
             sems, sems_s):
        c = lax.axis_index("c")
        s = lax.axis_index("s")
        wid = c * _SC_NS + s

        @pl.when(s < _SC_NS - 1)
        def _():
            pltpu.sync_copy(z_hbm, acc.at[pl.ds(s * stripe, stripe)])

        @pl.when(s == _SC_NS - 1)
        def _():
            pltpu.sync_copy(z_hbm.at[pl.ds(0, last)],
                            acc.at[pl.ds(s * stripe, last)])

        plsc.subcore_barrier()

        def block(b, carry):
            pltpu.sync_copy(src_hbm.at[wid, pl.ds(b * _NBE, _NBE)], idxs)
            pltpu.sync_copy(dst_hbm.at[wid, pl.ds(b * _NBE, _NBE)], idxd)
            _pipe_gather_scatter(h_hbm, idxs, idxd, _NBE, vals, sems, sems_s,
                                 acc, g=_GE)
            return carry

        lax.fori_loop(0, n_groups // _NBE, block, 0)
        plsc.subcore_barrier()

        @pl.when(s < _SC_NS - 1)
        def _():
            pltpu.sync_copy(acc.at[pl.ds(s * stripe, stripe)],
                            out_hbm.at[c, pl.ds(s * stripe, stripe)])

        @pl.when(s == _SC_NS - 1)
        def _():
            pltpu.sync_copy(acc.at[pl.ds(s * stripe, last)],
                            out_hbm.at[c, pl.ds(s * stripe, last)])

    f = pl.kernel(
        body,
        out_type=jax.ShapeDtypeStruct((_SC_NC, n, emb), jnp.float32),
        mesh=_sc_mesh(),
        compiler_params=pltpu.CompilerParams(use_tc_tiling_on_sc=False),
        scratch_types=[
            pltpu.VMEM_SHARED((n, emb), jnp.float32),
            pltpu.VMEM((_NBE, _GE), jnp.int32),
            pltpu.VMEM((_NBE, _GE), jnp.int32),
            pltpu.VMEM((2 * _GE, emb), jnp.float32),
            pltpu.SemaphoreType.DMA((2,)),
            pltpu.SemaphoreType.DMA((2,)),
        ],
    )
    return f(h, src3, dst3, z2)


def _add_split_body(a_ref, b_ref, o0, o1, o2, o3):
    y = a_ref[...] + b_ref[...]
    for q, o in enumerate((o0, o1, o2, o3)):
        o[...] = y[:, 32 * q:32 * (q + 1)]


def _add_split(a, b):
    m, n = a.shape
    grid = m // _BM
    return pl.pallas_call(
        _add_split_body,
        grid=(grid,),
        in_specs=[pl.BlockSpec((_BM, n), lambda i: (i, 0))] * 2,
        out_specs=[pl.BlockSpec((_BM, 32), lambda i: (i, 0))] * 4,
        out_shape=[jax.ShapeDtypeStruct((m, 32), jnp.float32)] * 4,
    )(a, b)


def _mm_relu_slabs_body(s0, s1, s2, s3, w0, w1, w2, w3, b_ref, o_ref):
    y = b_ref[...]
    for s_ref, w_ref in ((s0, w0), (s1, w1), (s2, w2), (s3, w3)):
        y = y + jnp.dot(s_ref[...], w_ref[...],
                        preferred_element_type=jnp.float32)
    o_ref[...] = jnp.maximum(y, 0.0)


def _mm_relu_slabs(slabs, w, b):
    m = slabs[0].shape[0]
    n = w.shape[1]
    w_slices = [w[32 * q:32 * (q + 1), :] for q in range(4)]
    grid = m // _BM
    return pl.pallas_call(
        _mm_relu_slabs_body,
        grid=(grid,),
        in_specs=[pl.BlockSpec((_BM, 32), lambda i: (i, 0))] * 4
        + [pl.BlockSpec((32, n), lambda i: (0, 0))] * 4
        + [pl.BlockSpec((n,), lambda i: (0,))],
        out_specs=pl.BlockSpec((_BM, n), lambda i: (i, 0)),
        out_shape=jax.ShapeDtypeStruct((m, n), jnp.float32),
    )(*slabs, *w_slices, b)


def _pipe_gather_scatter(table, idxs_v, idxd_v, n_groups, vals2, sems,
                         sems_s, acc, g=None):
    if g is None:
        g = _SC_G

    def gather_start(k, slot):
        pltpu.async_copy(table.at[idxs_v.at[k]],
                         vals2.at[pl.ds(slot * g, g)], sems.at[slot])

    def gather_wait(k, slot):
        pltpu.make_async_copy(table.at[idxs_v.at[k]],
                              vals2.at[pl.ds(slot * g, g)],
                              sems.at[slot]).wait()

    def scatter_start(k, slot):
        pltpu.async_copy(vals2.at[pl.ds(slot * g, g)], acc.at[idxd_v.at[k]],
                         sems_s.at[slot], add=True)

    def scatter_wait(k, slot):
        pltpu.make_async_copy(vals2.at[pl.ds(slot * g, g)],
                              acc.at[idxd_v.at[k]], sems_s.at[slot]).wait()

    gather_start(0, 0)

    def step(k, carry):
        slot = lax.rem(k, 2)
        nslot = 1 - slot

        @pl.when(k >= 1)
        def _():
            scatter_wait(k - 1, nslot)

        @pl.when(k + 1 < n_groups)
        def _():
            gather_start(k + 1, nslot)

        gather_wait(k, slot)
        scatter_start(k, slot)
        return carry

    lax.fori_loop(0, n_groups, step, 0)
    scatter_wait(n_groups - 1, (n_groups - 1) % 2)


def _sc_gather_xs(h, lab, nodeidx, nodelabel):
    n, emb = h.shape
    nsub = nodeidx.shape[0]
    n_groups = nsub // _SC_G
    nw = _SC_NC * _SC_NS
    k_max = (n_groups + nw - 1) // nw
    npad = nw * k_max
    ni2 = jnp.zeros((npad, _SC_G), jnp.int32)
    ni2 = ni2.at[:n_groups].set(nodeidx.reshape(n_groups, _SC_G))
    nl2 = jnp.zeros((npad, _SC_G), jnp.int32)
    nl2 = nl2.at[:n_groups].set(nodelabel.reshape(n_groups, _SC_G))
    g_ = _SC_G

    def body(h_hbm, lab_hbm, ni_hbm, nl_hbm, xh_hbm, xl_hbm,
             idx1, idx2, vals, vals2, semg, semg2, semw, semw2):
        c = lax.axis_index("c")
        s = lax.axis_index("s")
        wid = c * _SC_NS + s
        pltpu.sync_copy(ni_hbm.at[pl.ds(wid * k_max, k_max)], idx1)
        pltpu.sync_copy(nl_hbm.at[pl.ds(wid * k_max, k_max)], idx2)

        def gather_start(k, slot):
            pltpu.async_copy(h_hbm.at[idx1.at[k]],
                             vals.at[pl.ds(slot * g_, g_)], semg.at[slot])
            pltpu.async_copy(lab_hbm.at[idx2.at[k]],
                             vals2.at[pl.ds(slot * g_, g_)], semg2.at[slot])

        def gather_wait(k, slot):
            pltpu.make_async_copy(h_hbm.at[idx1.at[k]],
                                  vals.at[pl.ds(slot * g_, g_)],
                                  semg.at[slot]).wait()
            pltpu.make_async_copy(lab_hbm.at[idx2.at[k]],
                                  vals2.at[pl.ds(slot * g_, g_)],
                                  semg2.at[slot]).wait()

        def write_start(k, slot):
            base = (wid * k_max + k) * g_
            pltpu.async_copy(vals.at[pl.ds(slot * g_, g_)],
                             xh_hbm.at[pl.ds(base, g_)], semw.at[slot])
            pltpu.async_copy(vals2.at[pl.ds(slot * g_, g_)],
                             xl_hbm.at[pl.ds(base, g_)], semw2.at[slot])

        def write_wait(k, slot):
            base = (wid * k_max + k) * g_
            pltpu.make_async_copy(vals.at[pl.ds(slot * g_, g_)],
                                  xh_hbm.at[pl.ds(base, g_)],
                                  semw.at[slot]).wait()
            pltpu.make_async_copy(vals2.at[pl.ds(slot * g_, g_)],
                                  xl_hbm.at[pl.ds(base, g_)],
                                  semw2.at[slot]).wait()

        gather_start(0, 0)

        def step(k, carry):
            slot = lax.rem(k, 2)
            nslot = 1 - slot
            g = wid * k_max + k

            @pl.when((k >= 1) & (g - 1 < n_groups))
            def _():
                write_wait(k - 1, nslot)

            @pl.when((k + 1 < k_max) & (g + 1 < n_groups))
            def _():
                gather_start(k + 1, nslot)

            @pl.when(g < n_groups)
            def _():
                gather_wait(k, slot)
                write_start(k, slot)

            return carry

        lax.fori_loop(0, k_max, step, 0)
        last_g = wid * k_max + k_max - 1

        @pl.when(last_g < n_groups)
        def _():
            write_wait(k_max - 1, (k_max - 1) % 2)

    f = pl.kernel(
        body,
        out_type=(jax.ShapeDtypeStruct((nsub, emb), jnp.float32),
                  jax.ShapeDtypeStruct((nsub, emb), jnp.float32)),
        mesh=_sc_mesh(),
        scratch_types=[
            pltpu.VMEM((k_max, _SC_G), jnp.int32),
            pltpu.VMEM((k_max, _SC_G), jnp.int32),
            pltpu.VMEM((2 * _SC_G, emb), jnp.float32),
            pltpu.VMEM((2 * _SC_G, emb), jnp.float32),
            pltpu.SemaphoreType.DMA((2,)),
            pltpu.SemaphoreType.DMA((2,)),
            pltpu.SemaphoreType.DMA((2,)),
            pltpu.SemaphoreType.DMA((2,)),
        ],
    )
    return f(h, lab, ni2, nl2)


def _sc_segsum_sub(xs_slabs, src, dst, zeros):
    nsub = xs_slabs[0].shape[0]
    e = src.shape[0]
    e_per_s = e // _SC_NS
    n_groups = e_per_s // _GE
    stripe = ((nsub // _SC_NS + 7) // 8) * 8
    last = nsub - stripe * (_SC_NS - 1)
    z2 = zeros[: stripe * 32].reshape(stripe, 32)
    src3 = src.reshape(_SC_NS, n_groups, _GE)
    dst3 = dst.reshape(_SC_NS, n_groups, _GE)

    def body(x0, x1, x2, x3, src_hbm, dst_hbm, z_hbm, a0, a1, a2, a3,
             acc, idxs, idxd, vals, sems, sems_s):
        c = lax.axis_index("c")
        s = lax.axis_index("s")
        xs_t = (x0, x1, x2, x3)
        ag_t = (a0, a1, a2, a3)
        for qi in range(2):
            for cc in range(_SC_NC):
                @pl.when(c == cc)
                def _(qi=qi, cc=cc):
                    xq = xs_t[2 * cc + qi]
                    aq = ag_t[2 * cc + qi]

                    @pl.when(s < _SC_NS - 1)
                    def _():
                        pltpu.sync_copy(z_hbm,
                                        acc.at[pl.ds(s * stripe, stripe)])

                    @pl.when(s == _SC_NS - 1)
                    def _():
                        pltpu.sync_copy(z_hbm.at[pl.ds(0, last)],
                                        acc.at[pl.ds(s * stripe, last)])

                    plsc.subcore_barrier()

                    def block(b, carry, xq=xq):
                        pltpu.sync_copy(src_hbm.at[s, pl.ds(b * _NBE, _NBE)],
                                        idxs)
                        pltpu.sync_copy(dst_hbm.at[s, pl.ds(b * _NBE, _NBE)],
                                        idxd)
                        _pipe_gather_scatter(xq, idxs, idxd, _NBE,
                                             vals, sems, sems_s, acc, g=_GE)
                        return carry

                    lax.fori_loop(0, n_groups // _NBE, block, 0)
                    plsc.subcore_barrier()

                    @pl.when(s < _SC_NS - 1)
                    def _():
                        pltpu.sync_copy(acc.at[pl.ds(s * stripe, stripe)],
                                        aq.at[pl.ds(s * stripe, stripe)])

                    @pl.when(s == _SC_NS - 1)
                    def _():
                        pltpu.sync_copy(acc.at[pl.ds(s * stripe, last)],
                                        aq.at[pl.ds(s * stripe, last)])

                    plsc.subcore_barrier()

    f = pl.kernel(
        body,
        out_type=tuple(jax.ShapeDtypeStruct((nsub, 32), jnp.float32)
                       for _ in range(4)),
        mesh=_sc_mesh(),
        compiler_params=pltpu.CompilerParams(use_tc_tiling_on_sc=False),
        scratch_types=[
            pltpu.VMEM_SHARED((nsub, 32), jnp.float32),
            pltpu.VMEM((_NBE, _GE), jnp.int32),
            pltpu.VMEM((_NBE, _GE), jnp.int32),
            pltpu.VMEM((2 * _GE, 32), jnp.float32),
            pltpu.SemaphoreType.DMA((2,)),
            pltpu.SemaphoreType.DMA((2,)),
        ],
    )
    return f(*xs_slabs, src3, dst3, z2)


def _sc_scatter_mean(h_sub, nodeidx, zeros, n_out):
    nsub, emb = h_sub.shape
    n_groups = nsub // _SC_G
    k_max = (n_groups + _SC_NC * _SC_NS - 1) // (_SC_NC * _SC_NS)
    stripe = ((n_out // _SC_NS + 7) // 8) * 8
    last = n_out - stripe * (_SC_NS - 1)
    z_s = zeros[: stripe * emb].reshape(stripe, emb)
    ones = jnp.ones((_SC_G, emb), jnp.float32)

    def body(hs_hbm, ni_hbm, z_hbm, ones_hbm, sum_hbm, cnt_hbm,
             acc, idxd, vals, ones_v):
        c = lax.axis_index("c")
        s = lax.axis_index("s")
        wid = c * _SC_NS + s
        pltpu.sync_copy(ones_hbm, ones_v)

        def zero_acc():
            @pl.when(s < _SC_NS - 1)
            def _():
                pltpu.sync_copy(z_hbm, acc.at[pl.ds(s * stripe, stripe)])

            @pl.when(s == _SC_NS - 1)
            def _():
                pltpu.sync_copy(z_hbm.at[pl.ds(0, last)],
                                acc.at[pl.ds(s * stripe, last)])

        def flush_acc(dst_hbm):
            @pl.when(s < _SC_NS - 1)
            def _():
                pltpu.sync_copy(acc.at[pl.ds(s * stripe, stripe)],
                                dst_hbm.at[c, pl.ds(s * stripe, stripe)])

            @pl.when(s == _SC_NS - 1)
            def _():
                pltpu.sync_copy(acc.at[pl.ds(s * stripe, last)],
                                dst_hbm.at[c, pl.ds(s * stripe, last)])

        def scan_groups(do_group):
            def step(k, carry):
                g = wid * k_max + k

                @pl.when(g < n_groups)
                def _():
                    do_group(g * _SC_G)

                return carry

            lax.fori_loop(0, k_max, step, 0)

        zero_acc()
        plsc.subcore_barrier()

        def sum_group(base):
            pltpu.sync_copy(ni_hbm.at[pl.ds(base, _SC_G)], idxd)
            pltpu.sync_copy(hs_hbm.at[pl.ds(base, _SC_G)], vals)
            pltpu.sync_copy(vals, acc.at[idxd], add=True)

        scan_groups(sum_group)
        plsc.subcore_barrier()
        flush_acc(sum_hbm)
        plsc.subcore_barrier()

        zero_acc()
        plsc.subcore_barrier()

        def cnt_group(base):
            pltpu.sync_copy(ni_hbm.at[pl.ds(base, _SC_G)], idxd)
            pltpu.sync_copy(ones_v, acc.at[idxd], add=True)

        scan_groups(cnt_group)
        plsc.subcore_barrier()
        flush_acc(cnt_hbm)

    f = pl.kernel(
        body,
        out_type=(jax.ShapeDtypeStruct((_SC_NC, n_out, emb), jnp.float32),
                  jax.ShapeDtypeStruct((_SC_NC, n_out, emb), jnp.float32)),
        mesh=_sc_mesh(),
        scratch_types=[
            pltpu.VMEM_SHARED((n_out, emb), jnp.float32),
            pltpu.VMEM((_SC_G,), jnp.int32),
            pltpu.VMEM((_SC_G, emb), jnp.float32),
            pltpu.VMEM((_SC_G, emb), jnp.float32),
        ],
    )
    return f(h_sub, nodeidx, z_s, ones)


def kernel(x, edge_index, subg_nodeidx, subg_nodelabel, subg_edge_index, batch,
           W_enc, b_enc, label_emb, W_l, b_l, W_g, b_g, W_out, b_out):
    zeros = jnp.zeros((81920,), jnp.float32)

    h = _mm_relu([x], W_enc, b_enc)

    xh, xl = _sc_gather_xs(h, label_emb, subg_nodeidx, subg_nodelabel)
    xs_slabs = _add_split(xh, xl)

    agg_slabs = _sc_segsum_sub(xs_slabs, subg_edge_index[0],
                               subg_edge_index[1], zeros)

    h_sub = _mm_relu_slabs(agg_slabs, W_l, b_l)

    sums, cnts = _sc_scatter_mean(h_sub, subg_nodeidx, zeros, _N)
    h1 = _mean_residual(h, sums[0], sums[1], cnts[0], cnts[1])

    gp = _sc_segsum_edges(h1, edge_index[0], edge_index[1], zeros)

    h2 = _mm_relu([gp[0], gp[1]], W_g, b_g, residual=h1)

    return _pool_predict(h2, batch, W_out, b_out)


# speedup vs baseline: 5.6008x; 1.0362x over previous
import functools

import jax
import jax.numpy as jnp
from jax import lax
from jax.experimental import pallas as pl
from jax.experimental.pallas import tpu as pltpu
from jax.experimental.pallas import tpu_sc as plsc

_N = 10000
_E = 320000
_NSUB = 40000
_ESUB = 320000
_NGRAPH = 64
_EMB = 128

_BM = 2000


def _mm_relu_body(n_parts, residual, *refs):
    *x_refs, w_ref, b_ref, o_ref = refs
    acc = x_refs[0][...]
    for r in x_refs[1:n_parts]:
        acc = acc + r[...]
    y = jnp.dot(acc, w_ref[...], preferred_element_type=jnp.float32)
    y = jnp.maximum(y + b_ref[...], 0.0)
    if residual:
        y = x_refs[n_parts][...] + y
    o_ref[...] = y


def _mm_relu(parts, w, b, residual=None):
    m = parts[0].shape[0]
    k = parts[0].shape[1]
    n = w.shape[1]
    inputs = list(parts) + ([residual] if residual is not None else [])
    grid = m // _BM
    body = functools.partial(_mm_relu_body, len(parts), residual is not None)
    return pl.pallas_call(
        body,
        grid=(grid,),
        in_specs=[pl.BlockSpec((_BM, k), lambda i: (i, 0)) for _ in inputs]
        + [
            pl.BlockSpec((k, n), lambda i: (0, 0)),
            pl.BlockSpec((n,), lambda i: (0,)),
        ],
        out_specs=pl.BlockSpec((_BM, n), lambda i: (i, 0)),
        out_shape=jax.ShapeDtypeStruct((m, n), jnp.float32),
    )(*inputs, w, b)


def _mean_residual_body(h_ref, s0_ref, s1_ref, c0_ref, c1_ref, o_ref):
    cnt = jnp.maximum((c0_ref[...] + c1_ref[...])[:, 0:1], 1.0)
    o_ref[...] = h_ref[...] + (s0_ref[...] + s1_ref[...]) / cnt


def _mean_residual(h, s0, s1, c0, c1):
    m, n = h.shape
    grid = m // _BM
    return pl.pallas_call(
        _mean_residual_body,
        grid=(grid,),
        in_specs=[pl.BlockSpec((_BM, n), lambda i: (i, 0)) for _ in range(5)],
        out_specs=pl.BlockSpec((_BM, n), lambda i: (i, 0)),
        out_shape=jax.ShapeDtypeStruct((m, n), jnp.float32),
    )(h, s0, s1, c0, c1)


def _pool_body(h_ref, batch_ref, w_ref, b_ref, o_ref, acc_ref, cnt_ref):
    i = pl.program_id(0)

    @pl.when(i == 0)
    def _():
        acc_ref[...] = jnp.zeros_like(acc_ref)
        cnt_ref[...] = jnp.zeros_like(cnt_ref)

    seg = batch_ref[...].reshape(1, _BM)
    gids = lax.broadcasted_iota(jnp.int32, (_NGRAPH, _BM), 0)
    mask = (gids == seg).astype(jnp.float32)
    acc_ref[...] += jnp.dot(mask, h_ref[...], preferred_element_type=jnp.float32)
    cnt_ref[...] += jnp.sum(mask, axis=1, keepdims=True)

    @pl.when(i == pl.num_programs(0) - 1)
    def _():
        hg = acc_ref[...] / jnp.maximum(cnt_ref[...], 1.0)
        o_ref[...] = jnp.dot(hg, w_ref[...], preferred_element_type=jnp.float32) + b_ref[...]


def _pool_predict(h, batch, w_out, b_out):
    m, n = h.shape
    ntask = w_out.shape[1]
    w_pad = jnp.zeros((n, 128), jnp.float32).at[:, :ntask].set(w_out)
    b_pad = jnp.zeros((128,), jnp.float32).at[:ntask].set(b_out)
    batch3 = batch.reshape(m // _BM, 1, _BM)
    grid = m // _BM
    out = pl.pallas_call(
        _pool_body,
        grid=(grid,),
        in_specs=[
            pl.BlockSpec((_BM, n), lambda i: (i, 0)),
            pl.BlockSpec((1, 1, _BM), lambda i: (i, 0, 0)),
            pl.BlockSpec((n, 128), lambda i: (0, 0)),
            pl.BlockSpec((128,), lambda i: (0,)),
        ],
        out_specs=pl.BlockSpec((_NGRAPH, 128), lambda i: (0, 0)),
        out_shape=jax.ShapeDtypeStruct((_NGRAPH, 128), jnp.float32),
        scratch_shapes=[
            pltpu.VMEM((_NGRAPH, n), jnp.float32),
            pltpu.VMEM((_NGRAPH, 1), jnp.float32),
        ],
    )(h, batch3, w_pad, b_pad)
    return out[:, :ntask]


_SC_NC = 2
_SC_NS = 16
_SC_G = 80
_NB = 25
_GE = 125
_NBE = 20


def _sc_mesh():
    return plsc.VectorSubcoreMesh(core_axis_name="c", subcore_axis_name="s")


def _sc_segsum_edges(h, src, dst, zeros):
    n, emb = h.shape
    e = src.shape[0]
    e_per_w = e // (_SC_NC * _SC_NS)
    n_groups = e_per_w // _GE
    stripe = ((n // _SC_NS + 7) // 8) * 8
    last = n - stripe * (_SC_NS - 1)
    z2 = zeros[: stripe * emb].reshape(stripe, emb)
    nw = _SC_NC * _SC_NS
    src3 = src.reshape(nw, n_groups, _GE)
    dst3 = dst.reshape(nw, n_groups, _GE)

    def body(h_hbm, src_hbm, dst_hbm, z_hbm, out_hbm, acc, idxs, idxd, vals,
             sems, sems_s):
        c = lax.axis_index("c")
        s = lax.axis_index("s")
        wid = c * _SC_NS + s

        @pl.when(s < _SC_NS - 1)
        def _():
            pltpu.sync_copy(z_hbm, acc.at[pl.ds(s * stripe, stripe)])

        @pl.when(s == _SC_NS - 1)
        def _():
            pltpu.sync_copy(z_hbm.at[pl.ds(0, last)],
                            acc.at[pl.ds(s * stripe, last)])

        plsc.subcore_barrier()

        def block(b, carry):
            pltpu.sync_copy(src_hbm.at[wid, pl.ds(b * _NBE, _NBE)], idxs)
            pltpu.sync_copy(dst_hbm.at[wid, pl.ds(b * _NBE, _NBE)], idxd)
            _pipe_gather_scatter(h_hbm, idxs, idxd, _NBE, vals, sems, sems_s,
                                 acc, g=_GE)
            return carry

        lax.fori_loop(0, n_groups // _NBE, block, 0)
        plsc.subcore_barrier()

        @pl.when(s < _SC_NS - 1)
        def _():
            pltpu.sync_copy(acc.at[pl.ds(s * stripe, stripe)],
                            out_hbm.at[c, pl.ds(s * stripe, stripe)])

        @pl.when(s == _SC_NS - 1)
        def _():
            pltpu.sync_copy(acc.at[pl.ds(s * stripe, last)],
                            out_hbm.at[c, pl.ds(s * stripe, last)])

    f = pl.kernel(
        body,
        out_type=jax.ShapeDtypeStruct((_SC_NC, n, emb), jnp.float32),
        mesh=_sc_mesh(),
        compiler_params=pltpu.CompilerParams(use_tc_tiling_on_sc=False),
        scratch_types=[
            pltpu.VMEM_SHARED((n, emb), jnp.float32),
            pltpu.VMEM((_NBE, _GE), jnp.int32),
            pltpu.VMEM((_NBE, _GE), jnp.int32),
            pltpu.VMEM((2 * _GE, emb), jnp.float32),
            pltpu.SemaphoreType.DMA((2,)),
            pltpu.SemaphoreType.DMA((2,)),
        ],
    )
    return f(h, src3, dst3, z2)


def _add_split_body(a_ref, b_ref, o0, o1, o2, o3):
    y = a_ref[...] + b_ref[...]
    for q, o in enumerate((o0, o1, o2, o3)):
        o[...] = y[:, 32 * q:32 * (q + 1)]


def _add_split(a, b):
    m, n = a.shape
    grid = m // _BM
    return pl.pallas_call(
        _add_split_body,
        grid=(grid,),
        in_specs=[pl.BlockSpec((_BM, n), lambda i: (i, 0))] * 2,
        out_specs=[pl.BlockSpec((_BM, 32), lambda i: (i, 0))] * 4,
        out_shape=[jax.ShapeDtypeStruct((m, 32), jnp.float32)] * 4,
    )(a, b)


def _mm_relu_slabs_body(s0, s1, s2, s3, w0, w1, w2, w3, b_ref, o_ref):
    y = b_ref[...]
    for s_ref, w_ref in ((s0, w0), (s1, w1), (s2, w2), (s3, w3)):
        y = y + jnp.dot(s_ref[...], w_ref[...],
                        preferred_element_type=jnp.float32)
    o_ref[...] = jnp.maximum(y, 0.0)


def _mm_relu_slabs(slabs, w, b):
    m = slabs[0].shape[0]
    n = w.shape[1]
    w_slices = [w[32 * q:32 * (q + 1), :] for q in range(4)]
    grid = m // _BM
    return pl.pallas_call(
        _mm_relu_slabs_body,
        grid=(grid,),
        in_specs=[pl.BlockSpec((_BM, 32), lambda i: (i, 0))] * 4
        + [pl.BlockSpec((32, n), lambda i: (0, 0))] * 4
        + [pl.BlockSpec((n,), lambda i: (0,))],
        out_specs=pl.BlockSpec((_BM, n), lambda i: (i, 0)),
        out_shape=jax.ShapeDtypeStruct((m, n), jnp.float32),
    )(*slabs, *w_slices, b)


def _pipe_gather_scatter(table, idxs_v, idxd_v, n_groups, vals2, sems,
                         sems_s, acc, g=None):
    if g is None:
        g = _SC_G

    def gather_start(k, slot):
        pltpu.async_copy(table.at[idxs_v.at[k]],
                         vals2.at[pl.ds(slot * g, g)], sems.at[slot])

    def gather_wait(k, slot):
        pltpu.make_async_copy(table.at[idxs_v.at[k]],
                              vals2.at[pl.ds(slot * g, g)],
                              sems.at[slot]).wait()

    def scatter_start(k, slot):
        pltpu.async_copy(vals2.at[pl.ds(slot * g, g)], acc.at[idxd_v.at[k]],
                         sems_s.at[slot], add=True)

    def scatter_wait(k, slot):
        pltpu.make_async_copy(vals2.at[pl.ds(slot * g, g)],
                              acc.at[idxd_v.at[k]], sems_s.at[slot]).wait()

    gather_start(0, 0)

    def step(k, carry):
        slot = lax.rem(k, 2)
        nslot = 1 - slot

        @pl.when(k >= 1)
        def _():
            scatter_wait(k - 1, nslot)

        @pl.when(k + 1 < n_groups)
        def _():
            gather_start(k + 1, nslot)

        gather_wait(k, slot)
        scatter_start(k, slot)
        return carry

    lax.fori_loop(0, n_groups, step, 0)
    scatter_wait(n_groups - 1, (n_groups - 1) % 2)


def _sc_gather_xs(h, lab, nodeidx, nodelabel):
    n, emb = h.shape
    nsub = nodeidx.shape[0]
    n_groups = nsub // _SC_G
    nw = _SC_NC * _SC_NS
    k_max = (n_groups + nw - 1) // nw
    npad = nw * k_max
    ni2 = jnp.zeros((npad, _SC_G), jnp.int32)
    ni2 = ni2.at[:n_groups].set(nodeidx.reshape(n_groups, _SC_G))
    nl2 = jnp.zeros((npad, _SC_G), jnp.int32)
    nl2 = nl2.at[:n_groups].set(nodelabel.reshape(n_groups, _SC_G))
    g_ = _SC_G

    def body(h_hbm, lab_hbm, ni_hbm, nl_hbm, xh_hbm, xl_hbm,
             idx1, idx2, vals, vals2, semg, semg2, semw, semw2):
        c = lax.axis_index("c")
        s = lax.axis_index("s")
        wid = c * _SC_NS + s
        pltpu.sync_copy(ni_hbm.at[pl.ds(wid * k_max, k_max)], idx1)
        pltpu.sync_copy(nl_hbm.at[pl.ds(wid * k_max, k_max)], idx2)

        def gather_start(k, slot):
            pltpu.async_copy(h_hbm.at[idx1.at[k]],
                             vals.at[pl.ds(slot * g_, g_)], semg.at[slot])
            pltpu.async_copy(lab_hbm.at[idx2.at[k]],
                             vals2.at[pl.ds(slot * g_, g_)], semg2.at[slot])

        def gather_wait(k, slot):
            pltpu.make_async_copy(h_hbm.at[idx1.at[k]],
                                  vals.at[pl.ds(slot * g_, g_)],
                                  semg.at[slot]).wait()
            pltpu.make_async_copy(lab_hbm.at[idx2.at[k]],
                                  vals2.at[pl.ds(slot * g_, g_)],
                                  semg2.at[slot]).wait()

        def write_start(k, slot):
            base = (wid * k_max + k) * g_
            pltpu.async_copy(vals.at[pl.ds(slot * g_, g_)],
                             xh_hbm.at[pl.ds(base, g_)], semw.at[slot])
            pltpu.async_copy(vals2.at[pl.ds(slot * g_, g_)],
                             xl_hbm.at[pl.ds(base, g_)], semw2.at[slot])

        def write_wait(k, slot):
            base = (wid * k_max + k) * g_
            pltpu.make_async_copy(vals.at[pl.ds(slot * g_, g_)],
                                  xh_hbm.at[pl.ds(base, g_)],
                                  semw.at[slot]).wait()
            pltpu.make_async_copy(vals2.at[pl.ds(slot * g_, g_)],
                                  xl_hbm.at[pl.ds(base, g_)],
                                  semw2.at[slot]).wait()

        gather_start(0, 0)

        def step(k, carry):
            slot = lax.rem(k, 2)
            nslot = 1 - slot
            g = wid * k_max + k

            @pl.when((k >= 1) & (g - 1 < n_groups))
            def _():
                write_wait(k - 1, nslot)

            @pl.when((k + 1 < k_max) & (g + 1 < n_groups))
            def _():
                gather_start(k + 1, nslot)

            @pl.when(g < n_groups)
            def _():
                gather_wait(k, slot)
                write_start(k, slot)

            return carry

        lax.fori_loop(0, k_max, step, 0)
        last_g = wid * k_max + k_max - 1

        @pl.when(last_g < n_groups)
        def _():
            write_wait(k_max - 1, (k_max - 1) % 2)

    f = pl.kernel(
        body,
        out_type=(jax.ShapeDtypeStruct((nsub, emb), jnp.float32),
                  jax.ShapeDtypeStruct((nsub, emb), jnp.float32)),
        mesh=_sc_mesh(),
        scratch_types=[
            pltpu.VMEM((k_max, _SC_G), jnp.int32),
            pltpu.VMEM((k_max, _SC_G), jnp.int32),
            pltpu.VMEM((2 * _SC_G, emb), jnp.float32),
            pltpu.VMEM((2 * _SC_G, emb), jnp.float32),
            pltpu.SemaphoreType.DMA((2,)),
            pltpu.SemaphoreType.DMA((2,)),
            pltpu.SemaphoreType.DMA((2,)),
            pltpu.SemaphoreType.DMA((2,)),
        ],
    )
    return f(h, lab, ni2, nl2)


def _sc_segsum_sub(xs_slabs, src, dst, zeros):
    nsub = xs_slabs[0].shape[0]
    e = src.shape[0]
    e_per_s = e // _SC_NS
    n_groups = e_per_s // _GE
    stripe = ((nsub // _SC_NS + 7) // 8) * 8
    last = nsub - stripe * (_SC_NS - 1)
    z2 = zeros[: stripe * 32].reshape(stripe, 32)
    src3 = src.reshape(_SC_NS, n_groups, _GE)
    dst3 = dst.reshape(_SC_NS, n_groups, _GE)

    def body(x0, x1, x2, x3, src_hbm, dst_hbm, z_hbm, a0, a1, a2, a3,
             acc, idxs, idxd, vals, sems, sems_s):
        c = lax.axis_index("c")
        s = lax.axis_index("s")
        xs_t = (x0, x1, x2, x3)
        ag_t = (a0, a1, a2, a3)
        for qi in range(2):
            for cc in range(_SC_NC):
                @pl.when(c == cc)
                def _(qi=qi, cc=cc):
                    xq = xs_t[2 * cc + qi]
                    aq = ag_t[2 * cc + qi]

                    @pl.when(s < _SC_NS - 1)
                    def _():
                        pltpu.sync_copy(z_hbm,
                                        acc.at[pl.ds(s * stripe, stripe)])

                    @pl.when(s == _SC_NS - 1)
                    def _():
                        pltpu.sync_copy(z_hbm.at[pl.ds(0, last)],
                                        acc.at[pl.ds(s * stripe, last)])

                    plsc.subcore_barrier()

                    def block(b, carry, xq=xq):
                        pltpu.sync_copy(src_hbm.at[s, pl.ds(b * _NBE, _NBE)],
                                        idxs)
                        pltpu.sync_copy(dst_hbm.at[s, pl.ds(b * _NBE, _NBE)],
                                        idxd)
                        _pipe_gather_scatter(xq, idxs, idxd, _NBE,
                                             vals, sems, sems_s, acc, g=_GE)
                        return carry

                    lax.fori_loop(0, n_groups // _NBE, block, 0)
                    plsc.subcore_barrier()

                    @pl.when(s < _SC_NS - 1)
                    def _():
                        pltpu.sync_copy(acc.at[pl.ds(s * stripe, stripe)],
                                        aq.at[pl.ds(s * stripe, stripe)])

                    @pl.when(s == _SC_NS - 1)
                    def _():
                        pltpu.sync_copy(acc.at[pl.ds(s * stripe, last)],
                                        aq.at[pl.ds(s * stripe, last)])

                    plsc.subcore_barrier()

    f = pl.kernel(
        body,
        out_type=tuple(jax.ShapeDtypeStruct((nsub, 32), jnp.float32)
                       for _ in range(4)),
        mesh=_sc_mesh(),
        compiler_params=pltpu.CompilerParams(use_tc_tiling_on_sc=False),
        scratch_types=[
            pltpu.VMEM_SHARED((nsub, 32), jnp.float32),
            pltpu.VMEM((_NBE, _GE), jnp.int32),
            pltpu.VMEM((_NBE, _GE), jnp.int32),
            pltpu.VMEM((2 * _GE, 32), jnp.float32),
            pltpu.SemaphoreType.DMA((2,)),
            pltpu.SemaphoreType.DMA((2,)),
        ],
    )
    return f(*xs_slabs, src3, dst3, z2)


def _sc_scatter_mean(h_sub, nodeidx, zeros, n_out):
    nsub, emb = h_sub.shape
    n_groups = nsub // _GE
    nw = _SC_NC * _SC_NS
    k_max = n_groups // nw
    stripe = ((n_out // _SC_NS + 7) // 8) * 8
    last = n_out - stripe * (_SC_NS - 1)
    z_s = zeros[: stripe * emb].reshape(stripe, emb)
    ones = jnp.ones((_GE, emb), jnp.float32)
    ni2 = nodeidx.reshape(n_groups, _GE)

    def body(hs_hbm, ni_hbm, z_hbm, ones_hbm, sum_hbm, cnt_hbm,
             acc, idxd, vals, seml, sems):
        c = lax.axis_index("c")
        s = lax.axis_index("s")
        wid = c * _SC_NS + s
        pltpu.sync_copy(ni_hbm.at[pl.ds(wid * k_max, k_max)], idxd)

        def zero_acc():
            @pl.when(s < _SC_NS - 1)
            def _():
                pltpu.sync_copy(z_hbm, acc.at[pl.ds(s * stripe, stripe)])

            @pl.when(s == _SC_NS - 1)
            def _():
                pltpu.sync_copy(z_hbm.at[pl.ds(0, last)],
                                acc.at[pl.ds(s * stripe, last)])

        def flush_acc(dst_hbm):
            @pl.when(s < _SC_NS - 1)
            def _():
                pltpu.sync_copy(acc.at[pl.ds(s * stripe, stripe)],
                                dst_hbm.at[c, pl.ds(s * stripe, stripe)])

            @pl.when(s == _SC_NS - 1)
            def _():
                pltpu.sync_copy(acc.at[pl.ds(s * stripe, last)],
                                dst_hbm.at[c, pl.ds(s * stripe, last)])

        def load_start(k, slot):
            base = (wid * k_max + k) * _GE
            pltpu.async_copy(hs_hbm.at[pl.ds(base, _GE)],
                             vals.at[pl.ds(slot * _GE, _GE)], seml.at[slot])

        def load_wait(k, slot):
            base = (wid * k_max + k) * _GE
            pltpu.make_async_copy(hs_hbm.at[pl.ds(base, _GE)],
                                  vals.at[pl.ds(slot * _GE, _GE)],
                                  seml.at[slot]).wait()

        def scatter_start(k, slot, vslot):
            pltpu.async_copy(vals.at[pl.ds(vslot * _GE, _GE)],
                             acc.at[idxd.at[k]], sems.at[slot], add=True)

        def scatter_wait(k, slot, vslot):
            pltpu.make_async_copy(vals.at[pl.ds(vslot * _GE, _GE)],
                                  acc.at[idxd.at[k]], sems.at[slot]).wait()

        zero_acc()
        plsc.subcore_barrier()
        load_start(0, 0)

        def step(k, carry):
            slot = lax.rem(k, 2)
            nslot = 1 - slot

            @pl.when(k >= 1)
            def _():
                scatter_wait(k - 1, nslot, nslot)

            @pl.when(k + 1 < k_max)
            def _():
                load_start(k + 1, nslot)

            load_wait(k, slot)
            scatter_start(k, slot, slot)
            return carry

        lax.fori_loop(0, k_max, step, 0)
        scatter_wait(k_max - 1, (k_max - 1) % 2, (k_max - 1) % 2)
        plsc.subcore_barrier()
        flush_acc(sum_hbm)
        plsc.subcore_barrier()

        zero_acc()
        pltpu.sync_copy(ones_hbm, vals.at[pl.ds(0, _GE)])
        plsc.subcore_barrier()

        def cstep(k, carry):
            slot = lax.rem(k, 2)

            @pl.when(k >= 1)
            def _():
                scatter_wait(k - 1, 1 - slot, 0)

            scatter_start(k, slot, 0)
            return carry

        lax.fori_loop(0, k_max, cstep, 0)
        scatter_wait(k_max - 1, (k_max - 1) % 2, 0)
        plsc.subcore_barrier()
        flush_acc(cnt_hbm)

    f = pl.kernel(
        body,
        out_type=(jax.ShapeDtypeStruct((_SC_NC, n_out, emb), jnp.float32),
                  jax.ShapeDtypeStruct((_SC_NC, n_out, emb), jnp.float32)),
        mesh=_sc_mesh(),
        compiler_params=pltpu.CompilerParams(use_tc_tiling_on_sc=False),
        scratch_types=[
            pltpu.VMEM_SHARED((n_out, emb), jnp.float32),
            pltpu.VMEM((k_max, _GE), jnp.int32),
            pltpu.VMEM((2 * _GE, emb), jnp.float32),
            pltpu.SemaphoreType.DMA((2,)),
            pltpu.SemaphoreType.DMA((2,)),
        ],
    )
    return f(h_sub, ni2, z_s, ones)


def kernel(x, edge_index, subg_nodeidx, subg_nodelabel, subg_edge_index, batch,
           W_enc, b_enc, label_emb, W_l, b_l, W_g, b_g, W_out, b_out):
    zeros = jnp.zeros((81920,), jnp.float32)

    h = _mm_relu([x], W_enc, b_enc)

    xh, xl = _sc_gather_xs(h, label_emb, subg_nodeidx, subg_nodelabel)
    xs_slabs = _add_split(xh, xl)

    agg_slabs = _sc_segsum_sub(xs_slabs, subg_edge_index[0],
                               subg_edge_index[1], zeros)

    h_sub = _mm_relu_slabs(agg_slabs, W_l, b_l)

    sums, cnts = _sc_scatter_mean(h_sub, subg_nodeidx, zeros, _N)
    h1 = _mean_residual(h, sums[0], sums[1], cnts[0], cnts[1])

    gp = _sc_segsum_edges(h1, edge_index[0], edge_index[1], zeros)

    h2 = _mm_relu([gp[0], gp[1]], W_g, b_g, residual=h1)

    return _pool_predict(h2, batch, W_out, b_out)
